# Initial kernel scaffold; baseline (speedup 1.0000x reference)
import jax
import jax.numpy as jnp
from jax.experimental import pallas as pl


def kernel(in_feat, edge_index, W1, b1, W2, b2):
    raise NotImplementedError("write your pallas kernel here")


# speedup vs baseline: 3.9225x; 3.9225x over previous
import functools

import jax
import jax.numpy as jnp
from jax import lax
from jax.experimental import pallas as pl
from jax.experimental.pallas import tpu as pltpu
from jax.experimental.pallas import tpu_sc as plsc

N = 10000
E = 320000
D_IN = 128
D_H = 128
N_CLS = 40
D2 = 64

NC = 2
NS = 16
LANES = 16
NW = NC * NS

K = 128
NCHUNK = 80
EPW = NCHUNK * K
E_PAD = EPW * NW

N_PAD = 10240
RPT = N_PAD // NS
PAD_NODE = N_PAD - 1

BN = 256
_MESH = plsc.VectorSubcoreMesh(core_axis_name="c", subcore_axis_name="s")


def _deg_body(src_r, dst_r, dego, degi, src_v, dst_v, ones_v, zbuf,
              sdego, sdegi, sem):
    c = lax.axis_index("c")
    s = lax.axis_index("s")
    wid = s * NC + c

    def fill(i, _):
        ones_v[pl.ds(i * LANES, LANES)] = jnp.ones((LANES,), jnp.float32)
        zbuf[pl.ds(i * LANES, LANES)] = jnp.zeros((LANES,), jnp.float32)
        return 0

    lax.fori_loop(0, K // LANES, fill, 0)

    def zfill(i, _):
        zbuf[pl.ds(K + i * LANES, LANES)] = jnp.zeros((LANES,), jnp.float32)
        return 0

    lax.fori_loop(0, (RPT - K) // LANES, zfill, 0)
    pltpu.sync_copy(zbuf, sdego.at[pl.ds(s * RPT, RPT)])
    pltpu.sync_copy(zbuf, sdegi.at[pl.ds(s * RPT, RPT)])

    pltpu.sync_copy(src_r.at[pl.ds(wid * NCHUNK, NCHUNK)], src_v)
    pltpu.sync_copy(dst_r.at[pl.ds(wid * NCHUNK, NCHUNK)], dst_v)
    plsc.subcore_barrier()

    def step(j, _):
        pltpu.sync_copy(ones_v, sdego.at[src_v.at[j]], add=True)
        pltpu.sync_copy(ones_v, sdegi.at[dst_v.at[j]], add=True)
        return 0

    lax.fori_loop(0, NCHUNK, step, 0)
    plsc.subcore_barrier()
    pltpu.sync_copy(sdego.at[pl.ds(s * RPT, RPT)],
                    dego.at[c, pl.ds(s * RPT, RPT)])
    pltpu.sync_copy(sdegi.at[pl.ds(s * RPT, RPT)],
                    degi.at[c, pl.ds(s * RPT, RPT)])


_deg_kernel = functools.partial(
    pl.kernel,
    out_type=(
        jax.ShapeDtypeStruct((NC, N_PAD), jnp.float32),
        jax.ShapeDtypeStruct((NC, N_PAD), jnp.float32),
    ),
    mesh=_MESH,
    scratch_types=[
        pltpu.VMEM((NCHUNK, K), jnp.int32),
        pltpu.VMEM((NCHUNK, K), jnp.int32),
        pltpu.VMEM((K,), jnp.float32),
        pltpu.VMEM((RPT,), jnp.float32),
        pltpu.VMEM_SHARED((N_PAD,), jnp.float32),
        pltpu.VMEM_SHARED((N_PAD,), jnp.float32),
        pltpu.SemaphoreType.DMA,
    ],
)(_deg_body)


def _make_msg_kernel(d):
    def body(table, src_r, dst_r, out, src_v, dst_v, gbuf, acc, sem):
        c = lax.axis_index("c")
        s = lax.axis_index("s")
        wid = s * NC + c

        pltpu.sync_copy(src_r.at[pl.ds(wid * NCHUNK, NCHUNK)], src_v)
        pltpu.sync_copy(dst_r.at[pl.ds(wid * NCHUNK, NCHUNK)], dst_v)

        def zrow(i, _):
            def zlane(j, _):
                gbuf[i, pl.ds(j * LANES, LANES)] = jnp.zeros(
                    (LANES,), jnp.float32)
                return 0

            return lax.fori_loop(0, d // LANES, zlane, 0)

        lax.fori_loop(0, K, zrow, 0)

        def zcp(t, _):
            pltpu.sync_copy(gbuf, acc.at[pl.ds(s * RPT + t * K, K)])
            return 0

        lax.fori_loop(0, RPT // K, zcp, 0)
        plsc.subcore_barrier()

        def step(j, _):
            pltpu.async_copy(table.at[src_v.at[j]], gbuf, sem).wait()
            pltpu.sync_copy(gbuf, acc.at[dst_v.at[j]], add=True)
            return 0

        lax.fori_loop(0, NCHUNK, step, 0)
        plsc.subcore_barrier()
        pltpu.sync_copy(acc.at[pl.ds(s * RPT, RPT)],
                        out.at[c, pl.ds(s * RPT, RPT)])

    return functools.partial(
        pl.kernel,
        out_type=jax.ShapeDtypeStruct((NC, N_PAD, d), jnp.float32),
        mesh=_MESH,
        scratch_types=[
            pltpu.VMEM((NCHUNK, K), jnp.int32),
            pltpu.VMEM((NCHUNK, K), jnp.int32),
            pltpu.VMEM((K, d), jnp.float32),
            pltpu.VMEM_SHARED((N_PAD, d), jnp.float32),
            pltpu.SemaphoreType.DMA,
        ],
        compiler_params=pltpu.CompilerParams(
            use_tc_tiling_on_sc=False) if d < 128 else None,
    )(body)


_msg1_kernel = _make_msg_kernel(D_IN)
_msg2_kernel = _make_msg_kernel(D2)


def _norm_scale_body(x_ref, dego_ref, degi_ref, xs_ref, ns_ref, nd_ref):
    no = lax.rsqrt(jnp.maximum(dego_ref[0] + dego_ref[1], 1.0))
    nd = lax.rsqrt(jnp.maximum(degi_ref[0] + degi_ref[1], 1.0))
    xs_ref[...] = x_ref[...] * no
    ns_ref[...] = no
    nd_ref[...] = nd


def _norm_scale(x_pad, dego, degi):
    grid = (N_PAD // BN,)
    return pl.pallas_call(
        _norm_scale_body,
        grid=grid,
        in_specs=[
            pl.BlockSpec((BN, D_IN), lambda i: (i, 0)),
            pl.BlockSpec((NC, BN, 1), lambda i: (0, i, 0)),
            pl.BlockSpec((NC, BN, 1), lambda i: (0, i, 0)),
        ],
        out_specs=[
            pl.BlockSpec((BN, D_IN), lambda i: (i, 0)),
            pl.BlockSpec((BN, 1), lambda i: (i, 0)),
            pl.BlockSpec((BN, 1), lambda i: (i, 0)),
        ],
        out_shape=[
            jax.ShapeDtypeStruct((N_PAD, D_IN), jnp.float32),
            jax.ShapeDtypeStruct((N_PAD, 1), jnp.float32),
            jax.ShapeDtypeStruct((N_PAD, 1), jnp.float32),
        ],
    )(x_pad, dego.reshape(NC, N_PAD, 1), degi.reshape(NC, N_PAD, 1))


def _layer_body(agg_ref, w1_ref, b1_ref, ns_ref, nd_ref, w2_ref, m2_ref):
    agg = agg_ref[0] + agg_ref[1]
    h = jnp.dot(agg, w1_ref[...], preferred_element_type=jnp.float32)
    h = jnp.maximum(h * nd_ref[...] + b1_ref[...], 0.0)
    m2_ref[...] = jnp.dot(h * ns_ref[...], w2_ref[...],
                          preferred_element_type=jnp.float32)


def _layer(agg, w1, b1, ns, nd, w2p):
    grid = (N_PAD // BN,)
    return pl.pallas_call(
        _layer_body,
        grid=grid,
        in_specs=[
            pl.BlockSpec((NC, BN, D_IN), lambda i: (0, i, 0)),
            pl.BlockSpec((D_IN, D_H), lambda i: (0, 0)),
            pl.BlockSpec((1, D_H), lambda i: (0, 0)),
            pl.BlockSpec((BN, 1), lambda i: (i, 0)),
            pl.BlockSpec((BN, 1), lambda i: (i, 0)),
            pl.BlockSpec((D_H, D2), lambda i: (0, 0)),
        ],
        out_specs=pl.BlockSpec((BN, D2), lambda i: (i, 0)),
        out_shape=jax.ShapeDtypeStruct((N_PAD, D2), jnp.float32),
    )(agg, w1, b1.reshape(1, D_H), ns, nd, w2p)


def _final_body(agg_ref, nd_ref, b2_ref, out_ref):
    out_ref[...] = (agg_ref[0] + agg_ref[1]) * nd_ref[...] + b2_ref[...]


def _final(agg2, nd, b2p):
    grid = (N_PAD // BN,)
    return pl.pallas_call(
        _final_body,
        grid=grid,
        in_specs=[
            pl.BlockSpec((NC, BN, D2), lambda i: (0, i, 0)),
            pl.BlockSpec((BN, 1), lambda i: (i, 0)),
            pl.BlockSpec((1, D2), lambda i: (0, 0)),
        ],
        out_specs=pl.BlockSpec((BN, D2), lambda i: (i, 0)),
        out_shape=jax.ShapeDtypeStruct((N_PAD, D2), jnp.float32),
    )(agg2, nd, b2p)


def kernel(in_feat, edge_index, W1, b1, W2, b2):
    src = edge_index[0]
    dst = edge_index[1]
    pad = jnp.full((E_PAD - E,), PAD_NODE, jnp.int32)
    src_r = jnp.concatenate([src, pad]).reshape(NW * NCHUNK, K)
    dst_r = jnp.concatenate([dst, pad]).reshape(NW * NCHUNK, K)

    x_pad = jnp.pad(in_feat, ((0, N_PAD - N), (0, 0)))
    w2p = jnp.pad(W2, ((0, 0), (0, D2 - N_CLS)))
    b2p = jnp.pad(b2, (0, D2 - N_CLS)).reshape(1, D2)

    dego, degi = _deg_kernel(src_r, dst_r)
    xs, ns, nd = _norm_scale(x_pad, dego, degi)
    agg1 = _msg1_kernel(xs, src_r, dst_r)
    m2 = _layer(agg1, W1, b1, ns, nd, w2p)
    agg2 = _msg2_kernel(m2, src_r, dst_r)
    out = _final(agg2, nd, b2p)
    return out[:N, :N_CLS]


# speedup vs baseline: 4.1932x; 1.0690x over previous
import functools

import jax
import jax.numpy as jnp
from jax import lax
from jax.experimental import pallas as pl
from jax.experimental.pallas import tpu as pltpu
from jax.experimental.pallas import tpu_sc as plsc

N = 10000
E = 320000
D_IN = 128
D_H = 128
N_CLS = 40
D2 = 64

NC = 2
NS = 16
LANES = 16
NW = NC * NS

K = 128
NCHUNK = 80
EPW = NCHUNK * K
E_PAD = EPW * NW

N_PAD = 10240
RPT = N_PAD // NS
PAD_NODE = N_PAD - 1

BN = 256
_MESH = plsc.VectorSubcoreMesh(core_axis_name="c", subcore_axis_name="s")


def _deg_body(src_r, dst_r, dego, degi, src_v, dst_v, ones_v, zbuf,
              sdego, sdegi, sem):
    c = lax.axis_index("c")
    s = lax.axis_index("s")
    wid = s * NC + c

    def fill(i, _):
        ones_v[pl.ds(i * LANES, LANES)] = jnp.ones((LANES,), jnp.float32)
        zbuf[pl.ds(i * LANES, LANES)] = jnp.zeros((LANES,), jnp.float32)
        return 0

    lax.fori_loop(0, K // LANES, fill, 0)

    def zfill(i, _):
        zbuf[pl.ds(K + i * LANES, LANES)] = jnp.zeros((LANES,), jnp.float32)
        return 0

    lax.fori_loop(0, (RPT - K) // LANES, zfill, 0)
    pltpu.sync_copy(zbuf, sdego.at[pl.ds(s * RPT, RPT)])
    pltpu.sync_copy(zbuf, sdegi.at[pl.ds(s * RPT, RPT)])

    pltpu.sync_copy(src_r.at[pl.ds(wid * NCHUNK, NCHUNK)], src_v)
    pltpu.sync_copy(dst_r.at[pl.ds(wid * NCHUNK, NCHUNK)], dst_v)
    plsc.subcore_barrier()

    def step(j, _):
        pltpu.sync_copy(ones_v, sdego.at[src_v.at[j]], add=True)
        pltpu.sync_copy(ones_v, sdegi.at[dst_v.at[j]], add=True)
        return 0

    lax.fori_loop(0, NCHUNK, step, 0)
    plsc.subcore_barrier()
    pltpu.sync_copy(sdego.at[pl.ds(s * RPT, RPT)],
                    dego.at[c, pl.ds(s * RPT, RPT)])
    pltpu.sync_copy(sdegi.at[pl.ds(s * RPT, RPT)],
                    degi.at[c, pl.ds(s * RPT, RPT)])


_deg_kernel = functools.partial(
    pl.kernel,
    out_type=(
        jax.ShapeDtypeStruct((NC, N_PAD), jnp.float32),
        jax.ShapeDtypeStruct((NC, N_PAD), jnp.float32),
    ),
    mesh=_MESH,
    scratch_types=[
        pltpu.VMEM((NCHUNK, K), jnp.int32),
        pltpu.VMEM((NCHUNK, K), jnp.int32),
        pltpu.VMEM((K,), jnp.float32),
        pltpu.VMEM((RPT,), jnp.float32),
        pltpu.VMEM_SHARED((N_PAD,), jnp.float32),
        pltpu.VMEM_SHARED((N_PAD,), jnp.float32),
        pltpu.SemaphoreType.DMA,
    ],
)(_deg_body)


HCH = NCHUNK // 2


def _make_msg_kernel(d):
    def body(table, src_r, dst_r, out, src_v, dst_v,
             g0, g1, acc, gs0, gs1, ss0, ss1):
        gb = [g0, g1]
        gs = [gs0, gs1]
        ss = [ss0, ss1]
        c = lax.axis_index("c")
        s = lax.axis_index("s")
        wid = s * NC + c

        def zrow(i, _):
            def zlane(j, _):
                g0[i, pl.ds(j * LANES, LANES)] = jnp.zeros(
                    (LANES,), jnp.float32)
                return 0

            return lax.fori_loop(0, d // LANES, zlane, 0)

        lax.fori_loop(0, K, zrow, 0)

        def zcp(t, _):
            pltpu.sync_copy(g0, acc.at[pl.ds(s * RPT + t * K, K)])
            return 0

        lax.fori_loop(0, RPT // K, zcp, 0)
        plsc.subcore_barrier()

        def gd(j, t):
            return pltpu.make_async_copy(table.at[src_v.at[j]], gb[t], gs[t])

        def sd(j, t):
            return pltpu.make_async_copy(gb[t], acc.at[dst_v.at[j]], ss[t])

        for h in range(2):
            pltpu.sync_copy(
                src_r.at[pl.ds(wid * NCHUNK + h * HCH, HCH)], src_v)
            pltpu.sync_copy(
                dst_r.at[pl.ds(wid * NCHUNK + h * HCH, HCH)], dst_v)
            gd(0, 0).start()

            def step(i, _):
                for t in range(2):
                    j = i * 2 + t
                    gd(j, t).wait()
                    sd(j, t).start(add=True)

                    @pl.when(j + 1 < HCH)
                    def _next():
                        @pl.when(j >= 1)
                        def _drain():
                            sd(j - 1, 1 - t).wait()

                        gd(j + 1, 1 - t).start()

                return 0

            lax.fori_loop(0, HCH // 2, step, 0)
            sd(HCH - 2, 0).wait()
            sd(HCH - 1, 1).wait()

        plsc.subcore_barrier()
        pltpu.sync_copy(acc.at[pl.ds(s * RPT, RPT)],
                        out.at[c, pl.ds(s * RPT, RPT)])

    return functools.partial(
        pl.kernel,
        out_type=jax.ShapeDtypeStruct((NC, N_PAD, d), jnp.float32),
        mesh=_MESH,
        scratch_types=[
            pltpu.VMEM((HCH, K), jnp.int32),
            pltpu.VMEM((HCH, K), jnp.int32),
            pltpu.VMEM((K, d), jnp.float32),
            pltpu.VMEM((K, d), jnp.float32),
            pltpu.VMEM_SHARED((N_PAD, d), jnp.float32),
            pltpu.SemaphoreType.DMA,
            pltpu.SemaphoreType.DMA,
            pltpu.SemaphoreType.DMA,
            pltpu.SemaphoreType.DMA,
        ],
        compiler_params=pltpu.CompilerParams(
            use_tc_tiling_on_sc=False) if d < 128 else None,
    )(body)


_msg1_kernel = _make_msg_kernel(D_IN)
_msg2_kernel = _make_msg_kernel(D2)


def _norm_scale_body(x_ref, dego_ref, degi_ref, xs_ref, ns_ref, nd_ref):
    no = lax.rsqrt(jnp.maximum(dego_ref[0] + dego_ref[1], 1.0))
    nd = lax.rsqrt(jnp.maximum(degi_ref[0] + degi_ref[1], 1.0))
    xs_ref[...] = x_ref[...] * no
    ns_ref[...] = no
    nd_ref[...] = nd


def _norm_scale(x_pad, dego, degi):
    grid = (N_PAD // BN,)
    return pl.pallas_call(
        _norm_scale_body,
        grid=grid,
        in_specs=[
            pl.BlockSpec((BN, D_IN), lambda i: (i, 0)),
            pl.BlockSpec((NC, BN, 1), lambda i: (0, i, 0)),
            pl.BlockSpec((NC, BN, 1), lambda i: (0, i, 0)),
        ],
        out_specs=[
            pl.BlockSpec((BN, D_IN), lambda i: (i, 0)),
            pl.BlockSpec((BN, 1), lambda i: (i, 0)),
            pl.BlockSpec((BN, 1), lambda i: (i, 0)),
        ],
        out_shape=[
            jax.ShapeDtypeStruct((N_PAD, D_IN), jnp.float32),
            jax.ShapeDtypeStruct((N_PAD, 1), jnp.float32),
            jax.ShapeDtypeStruct((N_PAD, 1), jnp.float32),
        ],
    )(x_pad, dego.reshape(NC, N_PAD, 1), degi.reshape(NC, N_PAD, 1))


def _layer_body(agg_ref, w1_ref, b1_ref, ns_ref, nd_ref, w2_ref, m2_ref):
    agg = agg_ref[0] + agg_ref[1]
    h = jnp.dot(agg, w1_ref[...], preferred_element_type=jnp.float32)
    h = jnp.maximum(h * nd_ref[...] + b1_ref[...], 0.0)
    m2_ref[...] = jnp.dot(h * ns_ref[...], w2_ref[...],
                          preferred_element_type=jnp.float32)


def _layer(agg, w1, b1, ns, nd, w2p):
    grid = (N_PAD // BN,)
    return pl.pallas_call(
        _layer_body,
        grid=grid,
        in_specs=[
            pl.BlockSpec((NC, BN, D_IN), lambda i: (0, i, 0)),
            pl.BlockSpec((D_IN, D_H), lambda i: (0, 0)),
            pl.BlockSpec((1, D_H), lambda i: (0, 0)),
            pl.BlockSpec((BN, 1), lambda i: (i, 0)),
            pl.BlockSpec((BN, 1), lambda i: (i, 0)),
            pl.BlockSpec((D_H, D2), lambda i: (0, 0)),
        ],
        out_specs=pl.BlockSpec((BN, D2), lambda i: (i, 0)),
        out_shape=jax.ShapeDtypeStruct((N_PAD, D2), jnp.float32),
    )(agg, w1, b1.reshape(1, D_H), ns, nd, w2p)


def _final_body(agg_ref, nd_ref, b2_ref, out_ref):
    out_ref[...] = (agg_ref[0] + agg_ref[1]) * nd_ref[...] + b2_ref[...]


def _final(agg2, nd, b2p):
    grid = (N_PAD // BN,)
    return pl.pallas_call(
        _final_body,
        grid=grid,
        in_specs=[
            pl.BlockSpec((NC, BN, D2), lambda i: (0, i, 0)),
            pl.BlockSpec((BN, 1), lambda i: (i, 0)),
            pl.BlockSpec((1, D2), lambda i: (0, 0)),
        ],
        out_specs=pl.BlockSpec((BN, D2), lambda i: (i, 0)),
        out_shape=jax.ShapeDtypeStruct((N_PAD, D2), jnp.float32),
    )(agg2, nd, b2p)


def kernel(in_feat, edge_index, W1, b1, W2, b2):
    src = edge_index[0]
    dst = edge_index[1]
    pad = jnp.full((E_PAD - E,), PAD_NODE, jnp.int32)
    src_r = jnp.concatenate([src, pad]).reshape(NW * NCHUNK, K)
    dst_r = jnp.concatenate([dst, pad]).reshape(NW * NCHUNK, K)

    x_pad = jnp.pad(in_feat, ((0, N_PAD - N), (0, 0)))
    w2p = jnp.pad(W2, ((0, 0), (0, D2 - N_CLS)))
    b2p = jnp.pad(b2, (0, D2 - N_CLS)).reshape(1, D2)

    dego, degi = _deg_kernel(src_r, dst_r)
    xs, ns, nd = _norm_scale(x_pad, dego, degi)
    agg1 = _msg1_kernel(xs, src_r, dst_r)
    m2 = _layer(agg1, W1, b1, ns, nd, w2p)
    agg2 = _msg2_kernel(m2, src_r, dst_r)
    out = _final(agg2, nd, b2p)
    return out[:N, :N_CLS]


# speedup vs baseline: 8.9513x; 2.1347x over previous
import functools

import jax
import jax.numpy as jnp
from jax import lax
from jax.experimental import pallas as pl
from jax.experimental.pallas import tpu as pltpu
from jax.experimental.pallas import tpu_sc as plsc

N = 10000
E = 320000
D_IN = 128
D_H = 128
N_CLS = 40
D2 = 64

NC = 2
NS = 16
LANES = 16
NW = NC * NS

K = 128
NCHUNK = 80
EPW = NCHUNK * K
E_PAD = EPW * NW

N_PAD = 10240
RPT = N_PAD // NS
PAD_NODE = N_PAD - 1

BN = 256
_MESH = plsc.VectorSubcoreMesh(core_axis_name="c", subcore_axis_name="s")


def _deg_body(src_r, dst_r, dego, degi, src_v, dst_v, ones_v, zbuf,
              sdego, sdegi, sem):
    c = lax.axis_index("c")
    s = lax.axis_index("s")
    wid = s * NC + c

    def fill(i, _):
        ones_v[pl.ds(i * LANES, LANES)] = jnp.ones((LANES,), jnp.float32)
        zbuf[pl.ds(i * LANES, LANES)] = jnp.zeros((LANES,), jnp.float32)
        return 0

    lax.fori_loop(0, K // LANES, fill, 0)

    def zfill(i, _):
        zbuf[pl.ds(K + i * LANES, LANES)] = jnp.zeros((LANES,), jnp.float32)
        return 0

    lax.fori_loop(0, (RPT - K) // LANES, zfill, 0)
    pltpu.sync_copy(zbuf, sdego.at[pl.ds(s * RPT, RPT)])
    pltpu.sync_copy(zbuf, sdegi.at[pl.ds(s * RPT, RPT)])

    pltpu.sync_copy(src_r.at[pl.ds(wid * NCHUNK, NCHUNK)], src_v)
    pltpu.sync_copy(dst_r.at[pl.ds(wid * NCHUNK, NCHUNK)], dst_v)
    plsc.subcore_barrier()

    def step(j, _):
        pltpu.sync_copy(ones_v, sdego.at[src_v.at[j]], add=True)
        pltpu.sync_copy(ones_v, sdegi.at[dst_v.at[j]], add=True)
        return 0

    lax.fori_loop(0, NCHUNK, step, 0)
    plsc.subcore_barrier()
    pltpu.sync_copy(sdego.at[pl.ds(s * RPT, RPT)],
                    dego.at[c, pl.ds(s * RPT, RPT)])
    pltpu.sync_copy(sdegi.at[pl.ds(s * RPT, RPT)],
                    degi.at[c, pl.ds(s * RPT, RPT)])


_deg_kernel = functools.partial(
    pl.kernel,
    out_type=(
        jax.ShapeDtypeStruct((NC, N_PAD), jnp.float32),
        jax.ShapeDtypeStruct((NC, N_PAD), jnp.float32),
    ),
    mesh=_MESH,
    scratch_types=[
        pltpu.VMEM((NCHUNK, K), jnp.int32),
        pltpu.VMEM((NCHUNK, K), jnp.int32),
        pltpu.VMEM((K,), jnp.float32),
        pltpu.VMEM((RPT,), jnp.float32),
        pltpu.VMEM_SHARED((N_PAD,), jnp.float32),
        pltpu.VMEM_SHARED((N_PAD,), jnp.float32),
        pltpu.SemaphoreType.DMA,
    ],
)(_deg_body)


HCH = NCHUNK // 2


def _msg_body(table, src_r, dst_r, out, src_v, dst_v,
              g0, g1, stbl, acc, gs0, gs1, ss0, ss1):
    gb = [g0, g1]
    gs = [gs0, gs1]
    ss = [ss0, ss1]
    c = lax.axis_index("c")
    s = lax.axis_index("s")
    wid = s * NC + c

    pltpu.sync_copy(table.at[pl.ds(s * RPT, RPT)],
                    stbl.at[pl.ds(s * RPT, RPT)])

    def zrow(i, _):
        def zlane(j, _):
            g0[i, pl.ds(j * LANES, LANES)] = jnp.zeros((LANES,), jnp.float32)
            return 0

        return lax.fori_loop(0, D2 // LANES, zlane, 0)

    lax.fori_loop(0, K, zrow, 0)

    def zcp(t, _):
        pltpu.sync_copy(g0, acc.at[pl.ds(s * RPT + t * K, K)])
        return 0

    lax.fori_loop(0, RPT // K, zcp, 0)
    plsc.subcore_barrier()

    def gd(j, t):
        return pltpu.make_async_copy(stbl.at[src_v.at[j]], gb[t], gs[t])

    def sd(j, t):
        return pltpu.make_async_copy(gb[t], acc.at[dst_v.at[j]], ss[t])

    for h in range(2):
        pltpu.sync_copy(src_r.at[pl.ds(wid * NCHUNK + h * HCH, HCH)], src_v)
        pltpu.sync_copy(dst_r.at[pl.ds(wid * NCHUNK + h * HCH, HCH)], dst_v)
        gd(0, 0).start()

        def step(i, _):
            for t in range(2):
                j = i * 2 + t
                gd(j, t).wait()
                sd(j, t).start(add=True)

                @pl.when(j + 1 < HCH)
                def _next():
                    @pl.when(j >= 1)
                    def _drain():
                        sd(j - 1, 1 - t).wait()

                    gd(j + 1, 1 - t).start()

            return 0

        lax.fori_loop(0, HCH // 2, step, 0)
        sd(HCH - 2, 0).wait()
        sd(HCH - 1, 1).wait()

    plsc.subcore_barrier()
    pltpu.sync_copy(acc.at[pl.ds(s * RPT, RPT)],
                    out.at[c, pl.ds(s * RPT, RPT)])


_msg_kernel = functools.partial(
    pl.kernel,
    out_type=jax.ShapeDtypeStruct((NC, N_PAD, D2), jnp.float32),
    mesh=_MESH,
    scratch_types=[
        pltpu.VMEM((HCH, K), jnp.int32),
        pltpu.VMEM((HCH, K), jnp.int32),
        pltpu.VMEM((K, D2), jnp.float32),
        pltpu.VMEM((K, D2), jnp.float32),
        pltpu.VMEM_SHARED((N_PAD, D2), jnp.float32),
        pltpu.VMEM_SHARED((N_PAD, D2), jnp.float32),
        pltpu.SemaphoreType.DMA,
        pltpu.SemaphoreType.DMA,
        pltpu.SemaphoreType.DMA,
        pltpu.SemaphoreType.DMA,
    ],
    compiler_params=pltpu.CompilerParams(use_tc_tiling_on_sc=False),
)(_msg_body)


def _norm_scale_body(x_ref, dego_ref, degi_ref, xsa_ref, xsb_ref,
                     ns_ref, nd_ref):
    no = lax.rsqrt(jnp.maximum(dego_ref[0] + dego_ref[1], 1.0))
    nd = lax.rsqrt(jnp.maximum(degi_ref[0] + degi_ref[1], 1.0))
    xs = x_ref[...] * no
    xsa_ref[...] = xs[:, :D2]
    xsb_ref[...] = xs[:, D2:]
    ns_ref[...] = no
    nd_ref[...] = nd


def _norm_scale(x_pad, dego, degi):
    grid = (N_PAD // BN,)
    return pl.pallas_call(
        _norm_scale_body,
        grid=grid,
        in_specs=[
            pl.BlockSpec((BN, D_IN), lambda i: (i, 0)),
            pl.BlockSpec((NC, BN, 1), lambda i: (0, i, 0)),
            pl.BlockSpec((NC, BN, 1), lambda i: (0, i, 0)),
        ],
        out_specs=[
            pl.BlockSpec((BN, D2), lambda i: (i, 0)),
            pl.BlockSpec((BN, D2), lambda i: (i, 0)),
            pl.BlockSpec((BN, 1), lambda i: (i, 0)),
            pl.BlockSpec((BN, 1), lambda i: (i, 0)),
        ],
        out_shape=[
            jax.ShapeDtypeStruct((N_PAD, D2), jnp.float32),
            jax.ShapeDtypeStruct((N_PAD, D2), jnp.float32),
            jax.ShapeDtypeStruct((N_PAD, 1), jnp.float32),
            jax.ShapeDtypeStruct((N_PAD, 1), jnp.float32),
        ],
    )(x_pad, dego.reshape(NC, N_PAD, 1), degi.reshape(NC, N_PAD, 1))


def _layer_body(agga_ref, aggb_ref, w1_ref, b1_ref, ns_ref, nd_ref, w2_ref,
                m2_ref):
    a = agga_ref[0] + agga_ref[1]
    b = aggb_ref[0] + aggb_ref[1]
    h = (jnp.dot(a, w1_ref[:D2, :], preferred_element_type=jnp.float32)
         + jnp.dot(b, w1_ref[D2:, :], preferred_element_type=jnp.float32))
    h = jnp.maximum(h * nd_ref[...] + b1_ref[...], 0.0)
    m2_ref[...] = jnp.dot(h * ns_ref[...], w2_ref[...],
                          preferred_element_type=jnp.float32)


def _layer(agga, aggb, w1, b1, ns, nd, w2p):
    grid = (N_PAD // BN,)
    return pl.pallas_call(
        _layer_body,
        grid=grid,
        in_specs=[
            pl.BlockSpec((NC, BN, D2), lambda i: (0, i, 0)),
            pl.BlockSpec((NC, BN, D2), lambda i: (0, i, 0)),
            pl.BlockSpec((D_IN, D_H), lambda i: (0, 0)),
            pl.BlockSpec((1, D_H), lambda i: (0, 0)),
            pl.BlockSpec((BN, 1), lambda i: (i, 0)),
            pl.BlockSpec((BN, 1), lambda i: (i, 0)),
            pl.BlockSpec((D_H, D2), lambda i: (0, 0)),
        ],
        out_specs=pl.BlockSpec((BN, D2), lambda i: (i, 0)),
        out_shape=jax.ShapeDtypeStruct((N_PAD, D2), jnp.float32),
    )(agga, aggb, w1, b1.reshape(1, D_H), ns, nd, w2p)


def _final_body(agg_ref, nd_ref, b2_ref, out_ref):
    out_ref[...] = (agg_ref[0] + agg_ref[1]) * nd_ref[...] + b2_ref[...]


def _final(agg2, nd, b2p):
    grid = (N_PAD // BN,)
    return pl.pallas_call(
        _final_body,
        grid=grid,
        in_specs=[
            pl.BlockSpec((NC, BN, D2), lambda i: (0, i, 0)),
            pl.BlockSpec((BN, 1), lambda i: (i, 0)),
            pl.BlockSpec((1, D2), lambda i: (0, 0)),
        ],
        out_specs=pl.BlockSpec((BN, D2), lambda i: (i, 0)),
        out_shape=jax.ShapeDtypeStruct((N_PAD, D2), jnp.float32),
    )(agg2, nd, b2p)


def kernel(in_feat, edge_index, W1, b1, W2, b2):
    src = edge_index[0]
    dst = edge_index[1]
    pad = jnp.full((E_PAD - E,), PAD_NODE, jnp.int32)
    src_r = jnp.concatenate([src, pad]).reshape(NW * NCHUNK, K)
    dst_r = jnp.concatenate([dst, pad]).reshape(NW * NCHUNK, K)

    x_pad = jnp.pad(in_feat, ((0, N_PAD - N), (0, 0)))
    w2p = jnp.pad(W2, ((0, 0), (0, D2 - N_CLS)))
    b2p = jnp.pad(b2, (0, D2 - N_CLS)).reshape(1, D2)

    dego, degi = _deg_kernel(src_r, dst_r)
    xsa, xsb, ns, nd = _norm_scale(x_pad, dego, degi)
    agg1a = _msg_kernel(xsa, src_r, dst_r)
    agg1b = _msg_kernel(xsb, src_r, dst_r)
    m2 = _layer(agg1a, agg1b, W1, b1, ns, nd, w2p)
    agg2 = _msg_kernel(m2, src_r, dst_r)
    out = _final(agg2, nd, b2p)
    return out[:N, :N_CLS]


# speedup vs baseline: 9.3811x; 1.0480x over previous
import functools

import jax
import jax.numpy as jnp
from jax import lax
from jax.experimental import pallas as pl
from jax.experimental.pallas import tpu as pltpu
from jax.experimental.pallas import tpu_sc as plsc

N = 10000
E = 320000
D_IN = 128
D_H = 128
N_CLS = 40
D2 = 64

NC = 2
NS = 16
LANES = 16
NW = NC * NS

K = 128
NCHUNK = 80
EPW = NCHUNK * K
E_PAD = EPW * NW

N_PAD = 10240
RPT = N_PAD // NS
PAD_NODE = N_PAD - 1

BN = 256
_MESH = plsc.VectorSubcoreMesh(core_axis_name="c", subcore_axis_name="s")


def _deg_body(src_r, dst_r, dego, degi, src_v, dst_v, ones_v, zbuf,
              sdego, sdegi, semo, semi):
    c = lax.axis_index("c")
    s = lax.axis_index("s")
    wid = s * NC + c

    def fill(i, _):
        ones_v[pl.ds(i * LANES, LANES)] = jnp.ones((LANES,), jnp.float32)
        zbuf[pl.ds(i * LANES, LANES)] = jnp.zeros((LANES,), jnp.float32)
        return 0

    lax.fori_loop(0, K // LANES, fill, 0)

    def zfill(i, _):
        zbuf[pl.ds(K + i * LANES, LANES)] = jnp.zeros((LANES,), jnp.float32)
        return 0

    lax.fori_loop(0, (RPT - K) // LANES, zfill, 0)
    pltpu.sync_copy(zbuf, sdego.at[pl.ds(s * RPT, RPT)])
    pltpu.sync_copy(zbuf, sdegi.at[pl.ds(s * RPT, RPT)])

    pltpu.sync_copy(src_r.at[pl.ds(wid * NCHUNK, NCHUNK)], src_v)
    pltpu.sync_copy(dst_r.at[pl.ds(wid * NCHUNK, NCHUNK)], dst_v)
    plsc.subcore_barrier()

    def sdo(j):
        return pltpu.make_async_copy(ones_v, sdego.at[src_v.at[j]], semo)

    def sdi(j):
        return pltpu.make_async_copy(ones_v, sdegi.at[dst_v.at[j]], semi)

    def step(j, _):
        sdo(j).start(add=True)
        sdi(j).start(add=True)
        return 0

    lax.fori_loop(0, NCHUNK, step, 0)

    def dstep(j, _):
        sdo(j).wait()
        sdi(j).wait()
        return 0

    lax.fori_loop(0, NCHUNK, dstep, 0)
    plsc.subcore_barrier()
    pltpu.sync_copy(sdego.at[pl.ds(s * RPT, RPT)],
                    dego.at[c, pl.ds(s * RPT, RPT)])
    pltpu.sync_copy(sdegi.at[pl.ds(s * RPT, RPT)],
                    degi.at[c, pl.ds(s * RPT, RPT)])


_deg_kernel = functools.partial(
    pl.kernel,
    out_type=(
        jax.ShapeDtypeStruct((NC, N_PAD), jnp.float32),
        jax.ShapeDtypeStruct((NC, N_PAD), jnp.float32),
    ),
    mesh=_MESH,
    scratch_types=[
        pltpu.VMEM((NCHUNK, K), jnp.int32),
        pltpu.VMEM((NCHUNK, K), jnp.int32),
        pltpu.VMEM((K,), jnp.float32),
        pltpu.VMEM((RPT,), jnp.float32),
        pltpu.VMEM_SHARED((N_PAD,), jnp.float32),
        pltpu.VMEM_SHARED((N_PAD,), jnp.float32),
        pltpu.SemaphoreType.DMA,
        pltpu.SemaphoreType.DMA,
    ],
)(_deg_body)


HCH = NCHUNK // 2


def _msg_body(table, src_r, dst_r, out, src_v, dst_v,
              g0, g1, g2, g3, stbl, acc,
              gs0, gs1, gs2, gs3, ss0, ss1, ss2, ss3):
    gb = [g0, g1, g2, g3]
    gs = [gs0, gs1, gs2, gs3]
    ss = [ss0, ss1, ss2, ss3]
    c = lax.axis_index("c")
    s = lax.axis_index("s")
    wid = s * NC + c

    pltpu.sync_copy(table.at[pl.ds(s * RPT, RPT)],
                    stbl.at[pl.ds(s * RPT, RPT)])

    def zrow(i, _):
        def zlane(j, _):
            g0[i, pl.ds(j * LANES, LANES)] = jnp.zeros((LANES,), jnp.float32)
            return 0

        return lax.fori_loop(0, D2 // LANES, zlane, 0)

    lax.fori_loop(0, K, zrow, 0)

    def zcp(t, _):
        pltpu.sync_copy(g0, acc.at[pl.ds(s * RPT + t * K, K)])
        return 0

    lax.fori_loop(0, RPT // K, zcp, 0)
    plsc.subcore_barrier()

    def gd(j, t):
        return pltpu.make_async_copy(stbl.at[src_v.at[j]], gb[t], gs[t])

    def sd(j, t):
        return pltpu.make_async_copy(gb[t], acc.at[dst_v.at[j]], ss[t])

    for h in range(2):
        pltpu.sync_copy(src_r.at[pl.ds(wid * NCHUNK + h * HCH, HCH)], src_v)
        pltpu.sync_copy(dst_r.at[pl.ds(wid * NCHUNK + h * HCH, HCH)], dst_v)
        gd(0, 0).start()
        gd(1, 1).start()

        def step(i, _):
            for t in range(4):
                j = i * 4 + t
                gd(j, t).wait()
                sd(j, t).start(add=True)
                t2 = (t + 2) % 4

                @pl.when(j + 2 < HCH)
                def _next():
                    @pl.when(j >= 2)
                    def _drain():
                        sd(j - 2, t2).wait()

                    gd(j + 2, t2).start()

            return 0

        lax.fori_loop(0, HCH // 4, step, 0)
        for j in range(HCH - 4, HCH):
            sd(j, j % 4).wait()

    plsc.subcore_barrier()
    pltpu.sync_copy(acc.at[pl.ds(s * RPT, RPT)],
                    out.at[c, pl.ds(s * RPT, RPT)])


_msg_kernel = functools.partial(
    pl.kernel,
    out_type=jax.ShapeDtypeStruct((NC, N_PAD, D2), jnp.float32),
    mesh=_MESH,
    scratch_types=[
        pltpu.VMEM((HCH, K), jnp.int32),
        pltpu.VMEM((HCH, K), jnp.int32),
        pltpu.VMEM((K, D2), jnp.float32),
        pltpu.VMEM((K, D2), jnp.float32),
        pltpu.VMEM((K, D2), jnp.float32),
        pltpu.VMEM((K, D2), jnp.float32),
        pltpu.VMEM_SHARED((N_PAD, D2), jnp.float32),
        pltpu.VMEM_SHARED((N_PAD, D2), jnp.float32),
        pltpu.SemaphoreType.DMA,
        pltpu.SemaphoreType.DMA,
        pltpu.SemaphoreType.DMA,
        pltpu.SemaphoreType.DMA,
        pltpu.SemaphoreType.DMA,
        pltpu.SemaphoreType.DMA,
        pltpu.SemaphoreType.DMA,
        pltpu.SemaphoreType.DMA,
    ],
    compiler_params=pltpu.CompilerParams(use_tc_tiling_on_sc=False),
)(_msg_body)


def _norm_scale_body(x_ref, dego_ref, degi_ref, xsa_ref, xsb_ref,
                     ns_ref, nd_ref):
    no = lax.rsqrt(jnp.maximum(dego_ref[0] + dego_ref[1], 1.0))
    nd = lax.rsqrt(jnp.maximum(degi_ref[0] + degi_ref[1], 1.0))
    xs = x_ref[...] * no
    xsa_ref[...] = xs[:, :D2]
    xsb_ref[...] = xs[:, D2:]
    ns_ref[...] = no
    nd_ref[...] = nd


def _norm_scale(x_pad, dego, degi):
    grid = (N_PAD // BN,)
    return pl.pallas_call(
        _norm_scale_body,
        grid=grid,
        in_specs=[
            pl.BlockSpec((BN, D_IN), lambda i: (i, 0)),
            pl.BlockSpec((NC, BN, 1), lambda i: (0, i, 0)),
            pl.BlockSpec((NC, BN, 1), lambda i: (0, i, 0)),
        ],
        out_specs=[
            pl.BlockSpec((BN, D2), lambda i: (i, 0)),
            pl.BlockSpec((BN, D2), lambda i: (i, 0)),
            pl.BlockSpec((BN, 1), lambda i: (i, 0)),
            pl.BlockSpec((BN, 1), lambda i: (i, 0)),
        ],
        out_shape=[
            jax.ShapeDtypeStruct((N_PAD, D2), jnp.float32),
            jax.ShapeDtypeStruct((N_PAD, D2), jnp.float32),
            jax.ShapeDtypeStruct((N_PAD, 1), jnp.float32),
            jax.ShapeDtypeStruct((N_PAD, 1), jnp.float32),
        ],
    )(x_pad, dego.reshape(NC, N_PAD, 1), degi.reshape(NC, N_PAD, 1))


def _layer_body(agga_ref, aggb_ref, w1_ref, b1_ref, ns_ref, nd_ref, w2_ref,
                m2_ref):
    a = agga_ref[0] + agga_ref[1]
    b = aggb_ref[0] + aggb_ref[1]
    h = (jnp.dot(a, w1_ref[:D2, :], preferred_element_type=jnp.float32)
         + jnp.dot(b, w1_ref[D2:, :], preferred_element_type=jnp.float32))
    h = jnp.maximum(h * nd_ref[...] + b1_ref[...], 0.0)
    m2_ref[...] = jnp.dot(h * ns_ref[...], w2_ref[...],
                          preferred_element_type=jnp.float32)


def _layer(agga, aggb, w1, b1, ns, nd, w2p):
    grid = (N_PAD // BN,)
    return pl.pallas_call(
        _layer_body,
        grid=grid,
        in_specs=[
            pl.BlockSpec((NC, BN, D2), lambda i: (0, i, 0)),
            pl.BlockSpec((NC, BN, D2), lambda i: (0, i, 0)),
            pl.BlockSpec((D_IN, D_H), lambda i: (0, 0)),
            pl.BlockSpec((1, D_H), lambda i: (0, 0)),
            pl.BlockSpec((BN, 1), lambda i: (i, 0)),
            pl.BlockSpec((BN, 1), lambda i: (i, 0)),
            pl.BlockSpec((D_H, D2), lambda i: (0, 0)),
        ],
        out_specs=pl.BlockSpec((BN, D2), lambda i: (i, 0)),
        out_shape=jax.ShapeDtypeStruct((N_PAD, D2), jnp.float32),
    )(agga, aggb, w1, b1.reshape(1, D_H), ns, nd, w2p)


def _final_body(agg_ref, nd_ref, b2_ref, out_ref):
    out_ref[...] = (agg_ref[0] + agg_ref[1]) * nd_ref[...] + b2_ref[...]


def _final(agg2, nd, b2p):
    grid = (N_PAD // BN,)
    return pl.pallas_call(
        _final_body,
        grid=grid,
        in_specs=[
            pl.BlockSpec((NC, BN, D2), lambda i: (0, i, 0)),
            pl.BlockSpec((BN, 1), lambda i: (i, 0)),
            pl.BlockSpec((1, D2), lambda i: (0, 0)),
        ],
        out_specs=pl.BlockSpec((BN, D2), lambda i: (i, 0)),
        out_shape=jax.ShapeDtypeStruct((N_PAD, D2), jnp.float32),
    )(agg2, nd, b2p)


def kernel(in_feat, edge_index, W1, b1, W2, b2):
    src = edge_index[0]
    dst = edge_index[1]
    pad = jnp.full((E_PAD - E,), PAD_NODE, jnp.int32)
    src_r = jnp.concatenate([src, pad]).reshape(NW * NCHUNK, K)
    dst_r = jnp.concatenate([dst, pad]).reshape(NW * NCHUNK, K)

    x_pad = jnp.pad(in_feat, ((0, N_PAD - N), (0, 0)))
    w2p = jnp.pad(W2, ((0, 0), (0, D2 - N_CLS)))
    b2p = jnp.pad(b2, (0, D2 - N_CLS)).reshape(1, D2)

    dego, degi = _deg_kernel(src_r, dst_r)
    xsa, xsb, ns, nd = _norm_scale(x_pad, dego, degi)
    agg1a = _msg_kernel(xsa, src_r, dst_r)
    agg1b = _msg_kernel(xsb, src_r, dst_r)
    m2 = _layer(agg1a, agg1b, W1, b1, ns, nd, w2p)
    agg2 = _msg_kernel(m2, src_r, dst_r)
    out = _final(agg2, nd, b2p)
    return out[:N, :N_CLS]


# speedup vs baseline: 11.1460x; 1.1881x over previous
import functools

import jax
import jax.numpy as jnp
from jax import lax
from jax.experimental import pallas as pl
from jax.experimental.pallas import tpu as pltpu
from jax.experimental.pallas import tpu_sc as plsc

N = 10000
E = 320000
D_IN = 128
D_H = 128
N_CLS = 40
D2 = 64

NC = 2
NS = 16
LANES = 16
NW = NC * NS

K = 128
NCHUNK = 80
EPW = NCHUNK * K
E_PAD = EPW * NW

N_PAD = 10240
RPT = N_PAD // NS
PAD_NODE = N_PAD - 1

BN = 256
_MESH = plsc.VectorSubcoreMesh(core_axis_name="c", subcore_axis_name="s")


def _deg_body(src_r, dst_r, dego, degi, src_v, dst_v, ones_v, zbuf,
              sdego, sdegi, semo, semi):
    c = lax.axis_index("c")
    s = lax.axis_index("s")
    wid = s * NC + c

    def fill(i, _):
        ones_v[pl.ds(i * LANES, LANES)] = jnp.ones((LANES,), jnp.float32)
        zbuf[pl.ds(i * LANES, LANES)] = jnp.zeros((LANES,), jnp.float32)
        return 0

    lax.fori_loop(0, K // LANES, fill, 0)

    def zfill(i, _):
        zbuf[pl.ds(K + i * LANES, LANES)] = jnp.zeros((LANES,), jnp.float32)
        return 0

    lax.fori_loop(0, (RPT - K) // LANES, zfill, 0)
    pltpu.sync_copy(zbuf, sdego.at[pl.ds(s * RPT, RPT)])
    pltpu.sync_copy(zbuf, sdegi.at[pl.ds(s * RPT, RPT)])

    pltpu.sync_copy(src_r.at[pl.ds(wid * NCHUNK, NCHUNK)], src_v)
    pltpu.sync_copy(dst_r.at[pl.ds(wid * NCHUNK, NCHUNK)], dst_v)
    plsc.subcore_barrier()

    def sdo(j):
        return pltpu.make_async_copy(ones_v, sdego.at[src_v.at[j]], semo)

    def sdi(j):
        return pltpu.make_async_copy(ones_v, sdegi.at[dst_v.at[j]], semi)

    def step(j, _):
        sdo(j).start(add=True)
        sdi(j).start(add=True)
        return 0

    lax.fori_loop(0, NCHUNK, step, 0)

    def dstep(j, _):
        sdo(j).wait()
        sdi(j).wait()
        return 0

    lax.fori_loop(0, NCHUNK, dstep, 0)
    plsc.subcore_barrier()
    pltpu.sync_copy(sdego.at[pl.ds(s * RPT, RPT)],
                    dego.at[c, pl.ds(s * RPT, RPT)])
    pltpu.sync_copy(sdegi.at[pl.ds(s * RPT, RPT)],
                    degi.at[c, pl.ds(s * RPT, RPT)])


_deg_kernel = functools.partial(
    pl.kernel,
    out_type=(
        jax.ShapeDtypeStruct((NC, N_PAD), jnp.float32),
        jax.ShapeDtypeStruct((NC, N_PAD), jnp.float32),
    ),
    mesh=_MESH,
    scratch_types=[
        pltpu.VMEM((NCHUNK, K), jnp.int32),
        pltpu.VMEM((NCHUNK, K), jnp.int32),
        pltpu.VMEM((K,), jnp.float32),
        pltpu.VMEM((RPT,), jnp.float32),
        pltpu.VMEM_SHARED((N_PAD,), jnp.float32),
        pltpu.VMEM_SHARED((N_PAD,), jnp.float32),
        pltpu.SemaphoreType.DMA,
        pltpu.SemaphoreType.DMA,
    ],
    compiler_params=pltpu.CompilerParams(use_tc_tiling_on_sc=False),
)(_deg_body)


HCH = NCHUNK // 2


def _msg_body(table, src_r, dst_r, out, src_v, dst_v,
              g0, g1, g2, g3, stbl, acc,
              gs0, gs1, gs2, gs3, ss0, ss1, ss2, ss3):
    gb = [g0, g1, g2, g3]
    gs = [gs0, gs1, gs2, gs3]
    ss = [ss0, ss1, ss2, ss3]
    c = lax.axis_index("c")
    s = lax.axis_index("s")
    wid = s * NC + c

    pltpu.sync_copy(table.at[pl.ds(s * RPT, RPT)],
                    stbl.at[pl.ds(s * RPT, RPT)])

    def zrow(i, _):
        def zlane(j, _):
            g0[i, pl.ds(j * LANES, LANES)] = jnp.zeros((LANES,), jnp.float32)
            return 0

        return lax.fori_loop(0, D2 // LANES, zlane, 0)

    lax.fori_loop(0, K, zrow, 0)

    def zcp(t, _):
        pltpu.sync_copy(g0, acc.at[pl.ds(s * RPT + t * K, K)])
        return 0

    lax.fori_loop(0, RPT // K, zcp, 0)
    plsc.subcore_barrier()

    def gd(j, t):
        return pltpu.make_async_copy(stbl.at[src_v.at[j]], gb[t], gs[t])

    def sd(j, t):
        return pltpu.make_async_copy(gb[t], acc.at[dst_v.at[j]], ss[t])

    for h in range(2):
        pltpu.sync_copy(src_r.at[pl.ds(wid * NCHUNK + h * HCH, HCH)], src_v)
        pltpu.sync_copy(dst_r.at[pl.ds(wid * NCHUNK + h * HCH, HCH)], dst_v)
        gd(0, 0).start()
        gd(1, 1).start()

        def step(i, _):
            for t in range(4):
                j = i * 4 + t
                gd(j, t).wait()
                sd(j, t).start(add=True)
                t2 = (t + 2) % 4

                @pl.when(j + 2 < HCH)
                def _next():
                    @pl.when(j >= 2)
                    def _drain():
                        sd(j - 2, t2).wait()

                    gd(j + 2, t2).start()

            return 0

        lax.fori_loop(0, HCH // 4, step, 0)
        for j in range(HCH - 4, HCH):
            sd(j, j % 4).wait()

    plsc.subcore_barrier()
    pltpu.sync_copy(acc.at[pl.ds(s * RPT, RPT)],
                    out.at[c, pl.ds(s * RPT, RPT)])


_msg_kernel = functools.partial(
    pl.kernel,
    out_type=jax.ShapeDtypeStruct((NC, N_PAD, D2), jnp.float32),
    mesh=_MESH,
    scratch_types=[
        pltpu.VMEM((HCH, K), jnp.int32),
        pltpu.VMEM((HCH, K), jnp.int32),
        pltpu.VMEM((K, D2), jnp.float32),
        pltpu.VMEM((K, D2), jnp.float32),
        pltpu.VMEM((K, D2), jnp.float32),
        pltpu.VMEM((K, D2), jnp.float32),
        pltpu.VMEM_SHARED((N_PAD, D2), jnp.float32),
        pltpu.VMEM_SHARED((N_PAD, D2), jnp.float32),
        pltpu.SemaphoreType.DMA,
        pltpu.SemaphoreType.DMA,
        pltpu.SemaphoreType.DMA,
        pltpu.SemaphoreType.DMA,
        pltpu.SemaphoreType.DMA,
        pltpu.SemaphoreType.DMA,
        pltpu.SemaphoreType.DMA,
        pltpu.SemaphoreType.DMA,
    ],
    compiler_params=pltpu.CompilerParams(use_tc_tiling_on_sc=False),
)(_msg_body)


BT = 2048


def _norm_scale_body(x_ref, dego_ref, degi_ref, xsa_ref, xsb_ref,
                     ns_ref, nd_ref):
    no_r = lax.rsqrt(jnp.maximum(dego_ref[0:1, :] + dego_ref[1:2, :], 1.0))
    nd_r = lax.rsqrt(jnp.maximum(degi_ref[0:1, :] + degi_ref[1:2, :], 1.0))
    no = jnp.transpose(no_r)
    nd = jnp.transpose(nd_r)
    xs = x_ref[...] * no
    xsa_ref[...] = xs[:, :D2]
    xsb_ref[...] = xs[:, D2:]
    ns_ref[...] = no
    nd_ref[...] = nd


def _norm_scale(x_pad, dego, degi):
    grid = (N_PAD // BT,)
    return pl.pallas_call(
        _norm_scale_body,
        grid=grid,
        in_specs=[
            pl.BlockSpec((BT, D_IN), lambda i: (i, 0)),
            pl.BlockSpec((NC, BT), lambda i: (0, i)),
            pl.BlockSpec((NC, BT), lambda i: (0, i)),
        ],
        out_specs=[
            pl.BlockSpec((BT, D2), lambda i: (i, 0)),
            pl.BlockSpec((BT, D2), lambda i: (i, 0)),
            pl.BlockSpec((BT, 1), lambda i: (i, 0)),
            pl.BlockSpec((BT, 1), lambda i: (i, 0)),
        ],
        out_shape=[
            jax.ShapeDtypeStruct((N_PAD, D2), jnp.float32),
            jax.ShapeDtypeStruct((N_PAD, D2), jnp.float32),
            jax.ShapeDtypeStruct((N_PAD, 1), jnp.float32),
            jax.ShapeDtypeStruct((N_PAD, 1), jnp.float32),
        ],
    )(x_pad, dego, degi)


def _layer_body(agga_ref, aggb_ref, w1_ref, b1_ref, ns_ref, nd_ref, w2_ref,
                m2_ref):
    a = agga_ref[0] + agga_ref[1]
    b = aggb_ref[0] + aggb_ref[1]
    h = (jnp.dot(a, w1_ref[:D2, :], preferred_element_type=jnp.float32)
         + jnp.dot(b, w1_ref[D2:, :], preferred_element_type=jnp.float32))
    h = jnp.maximum(h * nd_ref[...] + b1_ref[...], 0.0)
    m2_ref[...] = jnp.dot(h * ns_ref[...], w2_ref[...],
                          preferred_element_type=jnp.float32)


def _layer(agga, aggb, w1, b1, ns, nd, w2p):
    grid = (N_PAD // BT,)
    return pl.pallas_call(
        _layer_body,
        grid=grid,
        in_specs=[
            pl.BlockSpec((NC, BT, D2), lambda i: (0, i, 0)),
            pl.BlockSpec((NC, BT, D2), lambda i: (0, i, 0)),
            pl.BlockSpec((D_IN, D_H), lambda i: (0, 0)),
            pl.BlockSpec((1, D_H), lambda i: (0, 0)),
            pl.BlockSpec((BT, 1), lambda i: (i, 0)),
            pl.BlockSpec((BT, 1), lambda i: (i, 0)),
            pl.BlockSpec((D_H, D2), lambda i: (0, 0)),
        ],
        out_specs=pl.BlockSpec((BT, D2), lambda i: (i, 0)),
        out_shape=jax.ShapeDtypeStruct((N_PAD, D2), jnp.float32),
    )(agga, aggb, w1, b1.reshape(1, D_H), ns, nd, w2p)


def _final_body(agg_ref, nd_ref, b2_ref, out_ref):
    out_ref[...] = (agg_ref[0] + agg_ref[1]) * nd_ref[...] + b2_ref[...]


def _final(agg2, nd, b2p):
    grid = (N_PAD // BT,)
    return pl.pallas_call(
        _final_body,
        grid=grid,
        in_specs=[
            pl.BlockSpec((NC, BT, D2), lambda i: (0, i, 0)),
            pl.BlockSpec((BT, 1), lambda i: (i, 0)),
            pl.BlockSpec((1, D2), lambda i: (0, 0)),
        ],
        out_specs=pl.BlockSpec((BT, D2), lambda i: (i, 0)),
        out_shape=jax.ShapeDtypeStruct((N_PAD, D2), jnp.float32),
    )(agg2, nd, b2p)


def kernel(in_feat, edge_index, W1, b1, W2, b2):
    src = edge_index[0]
    dst = edge_index[1]
    pad = jnp.full((E_PAD - E,), PAD_NODE, jnp.int32)
    src_r = jnp.concatenate([src, pad]).reshape(NW * NCHUNK, K)
    dst_r = jnp.concatenate([dst, pad]).reshape(NW * NCHUNK, K)

    x_pad = jnp.pad(in_feat, ((0, N_PAD - N), (0, 0)))
    w2p = jnp.pad(W2, ((0, 0), (0, D2 - N_CLS)))
    b2p = jnp.pad(b2, (0, D2 - N_CLS)).reshape(1, D2)

    dego, degi = _deg_kernel(src_r, dst_r)
    xsa, xsb, ns, nd = _norm_scale(x_pad, dego, degi)
    agg1a = _msg_kernel(xsa, src_r, dst_r)
    agg1b = _msg_kernel(xsb, src_r, dst_r)
    m2 = _layer(agg1a, agg1b, W1, b1, ns, nd, w2p)
    agg2 = _msg_kernel(m2, src_r, dst_r)
    out = _final(agg2, nd, b2p)
    return out[:N, :N_CLS]


# speedup vs baseline: 11.9398x; 1.0712x over previous
import functools

import jax
import jax.numpy as jnp
from jax import lax
from jax.experimental import pallas as pl
from jax.experimental.pallas import tpu as pltpu
from jax.experimental.pallas import tpu_sc as plsc

N = 10000
E = 320000
D_IN = 128
D_H = 128
N_CLS = 40
D2 = 64

NC = 2
NS = 16
LANES = 16
NW = NC * NS

K = 128
NCHUNK = 80
EPW = NCHUNK * K
E_PAD = EPW * NW

N_PAD = 10240
RPT = N_PAD // NS
PAD_NODE = N_PAD - 1

BN = 256
_MESH = plsc.VectorSubcoreMesh(core_axis_name="c", subcore_axis_name="s")


def _deg_body(src_r, dst_r, dego, degi, src_v, dst_v, ones_v, zbuf,
              sdego, sdegi, semo, semi):
    c = lax.axis_index("c")
    s = lax.axis_index("s")
    wid = s * NC + c

    def fill(i, _):
        ones_v[pl.ds(i * LANES, LANES)] = jnp.ones((LANES,), jnp.float32)
        zbuf[pl.ds(i * LANES, LANES)] = jnp.zeros((LANES,), jnp.float32)
        return 0

    lax.fori_loop(0, K // LANES, fill, 0)

    def zfill(i, _):
        zbuf[pl.ds(K + i * LANES, LANES)] = jnp.zeros((LANES,), jnp.float32)
        return 0

    lax.fori_loop(0, (RPT - K) // LANES, zfill, 0)
    pltpu.sync_copy(zbuf, sdego.at[pl.ds(s * RPT, RPT)])
    pltpu.sync_copy(zbuf, sdegi.at[pl.ds(s * RPT, RPT)])

    pltpu.sync_copy(src_r.at[pl.ds(wid * NCHUNK, NCHUNK)], src_v)
    pltpu.sync_copy(dst_r.at[pl.ds(wid * NCHUNK, NCHUNK)], dst_v)
    plsc.subcore_barrier()

    def sdo(j):
        return pltpu.make_async_copy(ones_v, sdego.at[src_v.at[j]], semo)

    def sdi(j):
        return pltpu.make_async_copy(ones_v, sdegi.at[dst_v.at[j]], semi)

    def step(j, _):
        sdo(j).start(add=True)
        sdi(j).start(add=True)
        return 0

    lax.fori_loop(0, NCHUNK, step, 0)

    def dstep(j, _):
        sdo(j).wait()
        sdi(j).wait()
        return 0

    lax.fori_loop(0, NCHUNK, dstep, 0)
    plsc.subcore_barrier()
    pltpu.sync_copy(sdego.at[pl.ds(s * RPT, RPT)],
                    dego.at[c, pl.ds(s * RPT, RPT)])
    pltpu.sync_copy(sdegi.at[pl.ds(s * RPT, RPT)],
                    degi.at[c, pl.ds(s * RPT, RPT)])


_deg_kernel = functools.partial(
    pl.kernel,
    out_type=(
        jax.ShapeDtypeStruct((NC, N_PAD), jnp.float32),
        jax.ShapeDtypeStruct((NC, N_PAD), jnp.float32),
    ),
    mesh=_MESH,
    scratch_types=[
        pltpu.VMEM((NCHUNK, K), jnp.int32),
        pltpu.VMEM((NCHUNK, K), jnp.int32),
        pltpu.VMEM((K,), jnp.float32),
        pltpu.VMEM((RPT,), jnp.float32),
        pltpu.VMEM_SHARED((N_PAD,), jnp.float32),
        pltpu.VMEM_SHARED((N_PAD,), jnp.float32),
        pltpu.SemaphoreType.DMA,
        pltpu.SemaphoreType.DMA,
    ],
    compiler_params=pltpu.CompilerParams(use_tc_tiling_on_sc=False),
)(_deg_body)


HCH = NCHUNK // 2


def _msg_body(table, src_r, dst_r, out, src_v, dst_v,
              g0, g1, g2, g3, stbl, acc,
              gs0, gs1, gs2, gs3, ss0, ss1, ss2, ss3):
    gb = [g0, g1, g2, g3]
    gs = [gs0, gs1, gs2, gs3]
    ss = [ss0, ss1, ss2, ss3]
    c = lax.axis_index("c")
    s = lax.axis_index("s")
    wid = s * NC + c

    pltpu.sync_copy(table.at[pl.ds(s * RPT, RPT)],
                    stbl.at[pl.ds(s * RPT, RPT)])

    def zrow(i, _):
        def zlane(j, _):
            g0[i, pl.ds(j * LANES, LANES)] = jnp.zeros((LANES,), jnp.float32)
            return 0

        return lax.fori_loop(0, D2 // LANES, zlane, 0)

    lax.fori_loop(0, K, zrow, 0)

    def zcp(t, _):
        pltpu.sync_copy(g0, acc.at[pl.ds(s * RPT + t * K, K)])
        return 0

    lax.fori_loop(0, RPT // K, zcp, 0)
    plsc.subcore_barrier()

    def gd(j, t):
        return pltpu.make_async_copy(stbl.at[src_v.at[j]], gb[t], gs[t])

    def sd(j, t):
        return pltpu.make_async_copy(gb[t], acc.at[dst_v.at[j]], ss[t])

    for h in range(2):
        pltpu.sync_copy(src_r.at[pl.ds(wid * NCHUNK + h * HCH, HCH)], src_v)
        pltpu.sync_copy(dst_r.at[pl.ds(wid * NCHUNK + h * HCH, HCH)], dst_v)
        gd(0, 0).start()
        gd(1, 1).start()

        def step(i, _):
            for t in range(4):
                j = i * 4 + t
                gd(j, t).wait()
                sd(j, t).start(add=True)
                t2 = (t + 2) % 4

                @pl.when(j + 2 < HCH)
                def _next():
                    @pl.when(j >= 2)
                    def _drain():
                        sd(j - 2, t2).wait()

                    gd(j + 2, t2).start()

            return 0

        lax.fori_loop(0, HCH // 4, step, 0)
        for j in range(HCH - 4, HCH):
            sd(j, j % 4).wait()

    plsc.subcore_barrier()
    pltpu.sync_copy(acc.at[pl.ds(s * RPT, RPT)],
                    out.at[c, pl.ds(s * RPT, RPT)])


_MSG_SCRATCH = [
    pltpu.VMEM((HCH, K), jnp.int32),
    pltpu.VMEM((HCH, K), jnp.int32),
    pltpu.VMEM((K, D2), jnp.float32),
    pltpu.VMEM((K, D2), jnp.float32),
    pltpu.VMEM((K, D2), jnp.float32),
    pltpu.VMEM((K, D2), jnp.float32),
    pltpu.VMEM_SHARED((N_PAD, D2), jnp.float32),
    pltpu.VMEM_SHARED((N_PAD, D2), jnp.float32),
    pltpu.SemaphoreType.DMA,
    pltpu.SemaphoreType.DMA,
    pltpu.SemaphoreType.DMA,
    pltpu.SemaphoreType.DMA,
    pltpu.SemaphoreType.DMA,
    pltpu.SemaphoreType.DMA,
    pltpu.SemaphoreType.DMA,
    pltpu.SemaphoreType.DMA,
]

_msg_kernel = functools.partial(
    pl.kernel,
    out_type=jax.ShapeDtypeStruct((NC, N_PAD, D2), jnp.float32),
    mesh=_MESH,
    scratch_types=_MSG_SCRATCH,
    compiler_params=pltpu.CompilerParams(use_tc_tiling_on_sc=False),
)(_msg_body)


def _msgab_body(tbls, src_r, dst_r, out, src_v, dst_v,
                g0, g1, g2, g3, stbl, acc,
                gs0, gs1, gs2, gs3, ss0, ss1, ss2, ss3):
    gb = [g0, g1, g2, g3]
    gs = [gs0, gs1, gs2, gs3]
    ss = [ss0, ss1, ss2, ss3]
    c = lax.axis_index("c")
    s = lax.axis_index("s")

    pltpu.sync_copy(tbls.at[c, pl.ds(s * RPT, RPT)],
                    stbl.at[pl.ds(s * RPT, RPT)])

    def zrow(i, _):
        def zlane(j, _):
            g0[i, pl.ds(j * LANES, LANES)] = jnp.zeros((LANES,), jnp.float32)
            return 0

        return lax.fori_loop(0, D2 // LANES, zlane, 0)

    lax.fori_loop(0, K, zrow, 0)

    def zcp(t, _):
        pltpu.sync_copy(g0, acc.at[pl.ds(s * RPT + t * K, K)])
        return 0

    lax.fori_loop(0, RPT // K, zcp, 0)
    plsc.subcore_barrier()

    def gd(j, t):
        return pltpu.make_async_copy(stbl.at[src_v.at[j]], gb[t], gs[t])

    def sd(j, t):
        return pltpu.make_async_copy(gb[t], acc.at[dst_v.at[j]], ss[t])

    for h in range(4):
        base = s * 2 * NCHUNK + h * HCH
        pltpu.sync_copy(src_r.at[pl.ds(base, HCH)], src_v)
        pltpu.sync_copy(dst_r.at[pl.ds(base, HCH)], dst_v)
        gd(0, 0).start()
        gd(1, 1).start()

        def step(i, _):
            for t in range(4):
                j = i * 4 + t
                gd(j, t).wait()
                sd(j, t).start(add=True)
                t2 = (t + 2) % 4

                @pl.when(j + 2 < HCH)
                def _next():
                    @pl.when(j >= 2)
                    def _drain():
                        sd(j - 2, t2).wait()

                    gd(j + 2, t2).start()

            return 0

        lax.fori_loop(0, HCH // 4, step, 0)
        for j in range(HCH - 4, HCH):
            sd(j, j % 4).wait()

    plsc.subcore_barrier()
    pltpu.sync_copy(acc.at[pl.ds(s * RPT, RPT)],
                    out.at[c, pl.ds(s * RPT, RPT)])


_msgab_kernel = functools.partial(
    pl.kernel,
    out_type=jax.ShapeDtypeStruct((NC, N_PAD, D2), jnp.float32),
    mesh=_MESH,
    scratch_types=_MSG_SCRATCH,
    compiler_params=pltpu.CompilerParams(use_tc_tiling_on_sc=False),
)(_msgab_body)


BT = 2048


def _norm_scale_body(x_ref, dego_ref, degi_ref, xsab_ref, ns_ref, nd_ref):
    no_r = lax.rsqrt(jnp.maximum(dego_ref[0:1, :] + dego_ref[1:2, :], 1.0))
    nd_r = lax.rsqrt(jnp.maximum(degi_ref[0:1, :] + degi_ref[1:2, :], 1.0))
    no = jnp.transpose(no_r)
    nd = jnp.transpose(nd_r)
    xs = x_ref[...] * no
    xsab_ref[0] = xs[:, :D2]
    xsab_ref[1] = xs[:, D2:]
    ns_ref[...] = no
    nd_ref[...] = nd


def _norm_scale(x_pad, dego, degi):
    grid = (N_PAD // BT,)
    return pl.pallas_call(
        _norm_scale_body,
        grid=grid,
        in_specs=[
            pl.BlockSpec((BT, D_IN), lambda i: (i, 0)),
            pl.BlockSpec((NC, BT), lambda i: (0, i)),
            pl.BlockSpec((NC, BT), lambda i: (0, i)),
        ],
        out_specs=[
            pl.BlockSpec((NC, BT, D2), lambda i: (0, i, 0)),
            pl.BlockSpec((BT, 1), lambda i: (i, 0)),
            pl.BlockSpec((BT, 1), lambda i: (i, 0)),
        ],
        out_shape=[
            jax.ShapeDtypeStruct((NC, N_PAD, D2), jnp.float32),
            jax.ShapeDtypeStruct((N_PAD, 1), jnp.float32),
            jax.ShapeDtypeStruct((N_PAD, 1), jnp.float32),
        ],
    )(x_pad, dego, degi)


def _layer_body(agg_ref, w1_ref, b1_ref, ns_ref, nd_ref, w2_ref, m2_ref):
    h = (jnp.dot(agg_ref[0], w1_ref[:D2, :],
                 preferred_element_type=jnp.float32)
         + jnp.dot(agg_ref[1], w1_ref[D2:, :],
                   preferred_element_type=jnp.float32))
    h = jnp.maximum(h * nd_ref[...] + b1_ref[...], 0.0)
    m2_ref[...] = jnp.dot(h * ns_ref[...], w2_ref[...],
                          preferred_element_type=jnp.float32)


def _layer(agg, w1, b1, ns, nd, w2p):
    grid = (N_PAD // BT,)
    return pl.pallas_call(
        _layer_body,
        grid=grid,
        in_specs=[
            pl.BlockSpec((NC, BT, D2), lambda i: (0, i, 0)),
            pl.BlockSpec((D_IN, D_H), lambda i: (0, 0)),
            pl.BlockSpec((1, D_H), lambda i: (0, 0)),
            pl.BlockSpec((BT, 1), lambda i: (i, 0)),
            pl.BlockSpec((BT, 1), lambda i: (i, 0)),
            pl.BlockSpec((D_H, D2), lambda i: (0, 0)),
        ],
        out_specs=pl.BlockSpec((BT, D2), lambda i: (i, 0)),
        out_shape=jax.ShapeDtypeStruct((N_PAD, D2), jnp.float32),
    )(agg, w1, b1.reshape(1, D_H), ns, nd, w2p)


def _final_body(agg_ref, nd_ref, b2_ref, out_ref):
    res = (agg_ref[0] + agg_ref[1]) * nd_ref[...] + b2_ref[...]
    out_ref[...] = res[:, :N_CLS]


def _final(agg2, nd, b2p):
    grid = (N_PAD // BT,)
    return pl.pallas_call(
        _final_body,
        grid=grid,
        in_specs=[
            pl.BlockSpec((NC, BT, D2), lambda i: (0, i, 0)),
            pl.BlockSpec((BT, 1), lambda i: (i, 0)),
            pl.BlockSpec((1, D2), lambda i: (0, 0)),
        ],
        out_specs=pl.BlockSpec((BT, N_CLS), lambda i: (i, 0)),
        out_shape=jax.ShapeDtypeStruct((N, N_CLS), jnp.float32),
    )(agg2, nd, b2p)


def kernel(in_feat, edge_index, W1, b1, W2, b2):
    src = edge_index[0]
    dst = edge_index[1]
    pad = jnp.full((E_PAD - E,), PAD_NODE, jnp.int32)
    src_r = jnp.concatenate([src, pad]).reshape(NW * NCHUNK, K)
    dst_r = jnp.concatenate([dst, pad]).reshape(NW * NCHUNK, K)

    x_pad = jnp.pad(in_feat, ((0, N_PAD - N), (0, 0)))
    w2p = jnp.pad(W2, ((0, 0), (0, D2 - N_CLS)))
    b2p = jnp.pad(b2, (0, D2 - N_CLS)).reshape(1, D2)

    dego, degi = _deg_kernel(src_r, dst_r)
    xsab, ns, nd = _norm_scale(x_pad, dego, degi)
    agg1 = _msgab_kernel(xsab, src_r, dst_r)
    m2 = _layer(agg1, W1, b1, ns, nd, w2p)
    agg2 = _msg_kernel(m2, src_r, dst_r)
    return _final(agg2, nd, b2p)


# speedup vs baseline: 12.4319x; 1.0412x over previous
import functools

import jax
import jax.numpy as jnp
from jax import lax
from jax.experimental import pallas as pl
from jax.experimental.pallas import tpu as pltpu
from jax.experimental.pallas import tpu_sc as plsc

N = 10000
E = 320000
D_IN = 128
D_H = 128
N_CLS = 40
DH = 64
D2 = 48

NC = 2
NS = 16
LANES = 16
NW = NC * NS

K = 128
NCHUNK = 80
EPW = NCHUNK * K
E_PAD = EPW * NW

N_PAD = 10240
RPT = N_PAD // NS
PAD_NODE = N_PAD - 1

BN = 256
_MESH = plsc.VectorSubcoreMesh(core_axis_name="c", subcore_axis_name="s")


def _deg_body(src_r, dst_r, dego, degi, src_v, dst_v, ones_v, zbuf,
              sdego, sdegi, semo, semi):
    c = lax.axis_index("c")
    s = lax.axis_index("s")
    wid = s * NC + c

    def fill(i, _):
        ones_v[pl.ds(i * LANES, LANES)] = jnp.ones((LANES,), jnp.float32)
        zbuf[pl.ds(i * LANES, LANES)] = jnp.zeros((LANES,), jnp.float32)
        return 0

    lax.fori_loop(0, K // LANES, fill, 0)

    def zfill(i, _):
        zbuf[pl.ds(K + i * LANES, LANES)] = jnp.zeros((LANES,), jnp.float32)
        return 0

    lax.fori_loop(0, (RPT - K) // LANES, zfill, 0)
    pltpu.sync_copy(zbuf, sdego.at[pl.ds(s * RPT, RPT)])
    pltpu.sync_copy(zbuf, sdegi.at[pl.ds(s * RPT, RPT)])

    pltpu.sync_copy(src_r.at[pl.ds(wid * NCHUNK, NCHUNK)], src_v)
    pltpu.sync_copy(dst_r.at[pl.ds(wid * NCHUNK, NCHUNK)], dst_v)
    plsc.subcore_barrier()

    def sdo(j):
        return pltpu.make_async_copy(ones_v, sdego.at[src_v.at[j]], semo)

    def sdi(j):
        return pltpu.make_async_copy(ones_v, sdegi.at[dst_v.at[j]], semi)

    def step(j, _):
        sdo(j).start(add=True)
        sdi(j).start(add=True)
        return 0

    lax.fori_loop(0, NCHUNK, step, 0)

    def dstep(j, _):
        sdo(j).wait()
        sdi(j).wait()
        return 0

    lax.fori_loop(0, NCHUNK, dstep, 0)
    plsc.subcore_barrier()
    pltpu.sync_copy(sdego.at[pl.ds(s * RPT, RPT)],
                    dego.at[c, pl.ds(s * RPT, RPT)])
    pltpu.sync_copy(sdegi.at[pl.ds(s * RPT, RPT)],
                    degi.at[c, pl.ds(s * RPT, RPT)])


_deg_kernel = functools.partial(
    pl.kernel,
    out_type=(
        jax.ShapeDtypeStruct((NC, N_PAD), jnp.float32),
        jax.ShapeDtypeStruct((NC, N_PAD), jnp.float32),
    ),
    mesh=_MESH,
    scratch_types=[
        pltpu.VMEM((NCHUNK, K), jnp.int32),
        pltpu.VMEM((NCHUNK, K), jnp.int32),
        pltpu.VMEM((K,), jnp.float32),
        pltpu.VMEM((RPT,), jnp.float32),
        pltpu.VMEM_SHARED((N_PAD,), jnp.float32),
        pltpu.VMEM_SHARED((N_PAD,), jnp.float32),
        pltpu.SemaphoreType.DMA,
        pltpu.SemaphoreType.DMA,
    ],
    compiler_params=pltpu.CompilerParams(use_tc_tiling_on_sc=False),
)(_deg_body)


HCH = NCHUNK // 2


def _msg_body(table, src_r, dst_r, out, src_v, dst_v,
              g0, g1, g2, g3, stbl, acc,
              gs0, gs1, gs2, gs3, ss0, ss1, ss2, ss3):
    gb = [g0, g1, g2, g3]
    gs = [gs0, gs1, gs2, gs3]
    ss = [ss0, ss1, ss2, ss3]
    c = lax.axis_index("c")
    s = lax.axis_index("s")
    wid = s * NC + c

    pltpu.sync_copy(table.at[pl.ds(s * RPT, RPT)],
                    stbl.at[pl.ds(s * RPT, RPT)])

    def zrow(i, _):
        def zlane(j, _):
            g0[i, pl.ds(j * LANES, LANES)] = jnp.zeros((LANES,), jnp.float32)
            return 0

        return lax.fori_loop(0, D2 // LANES, zlane, 0)

    lax.fori_loop(0, K, zrow, 0)

    def zcp(t, _):
        pltpu.sync_copy(g0, acc.at[pl.ds(s * RPT + t * K, K)])
        return 0

    lax.fori_loop(0, RPT // K, zcp, 0)
    plsc.subcore_barrier()

    def gd(j, t):
        return pltpu.make_async_copy(stbl.at[src_v.at[j]], gb[t], gs[t])

    def sd(j, t):
        return pltpu.make_async_copy(gb[t], acc.at[dst_v.at[j]], ss[t])

    for h in range(2):
        pltpu.sync_copy(src_r.at[pl.ds(wid * NCHUNK + h * HCH, HCH)], src_v)
        pltpu.sync_copy(dst_r.at[pl.ds(wid * NCHUNK + h * HCH, HCH)], dst_v)
        gd(0, 0).start()
        gd(1, 1).start()

        def step(i, _):
            for t in range(4):
                j = i * 4 + t
                gd(j, t).wait()
                sd(j, t).start(add=True)
                t2 = (t + 2) % 4

                @pl.when(j + 2 < HCH)
                def _next():
                    @pl.when(j >= 2)
                    def _drain():
                        sd(j - 2, t2).wait()

                    gd(j + 2, t2).start()

            return 0

        lax.fori_loop(0, HCH // 4, step, 0)
        for j in range(HCH - 4, HCH):
            sd(j, j % 4).wait()

    plsc.subcore_barrier()
    pltpu.sync_copy(acc.at[pl.ds(s * RPT, RPT)],
                    out.at[c, pl.ds(s * RPT, RPT)])


def _msg_scratch(w):
    return [
        pltpu.VMEM((HCH, K), jnp.int32),
        pltpu.VMEM((HCH, K), jnp.int32),
        pltpu.VMEM((K, w), jnp.float32),
        pltpu.VMEM((K, w), jnp.float32),
        pltpu.VMEM((K, w), jnp.float32),
        pltpu.VMEM((K, w), jnp.float32),
        pltpu.VMEM_SHARED((N_PAD, w), jnp.float32),
        pltpu.VMEM_SHARED((N_PAD, w), jnp.float32),
        pltpu.SemaphoreType.DMA,
        pltpu.SemaphoreType.DMA,
        pltpu.SemaphoreType.DMA,
        pltpu.SemaphoreType.DMA,
        pltpu.SemaphoreType.DMA,
        pltpu.SemaphoreType.DMA,
        pltpu.SemaphoreType.DMA,
        pltpu.SemaphoreType.DMA,
    ]


_msg_kernel = functools.partial(
    pl.kernel,
    out_type=jax.ShapeDtypeStruct((NC, N_PAD, D2), jnp.float32),
    mesh=_MESH,
    scratch_types=_msg_scratch(D2),
    compiler_params=pltpu.CompilerParams(use_tc_tiling_on_sc=False),
)(_msg_body)


def _msgab_body(tbls, src_r, dst_r, out, src_v, dst_v,
                g0, g1, g2, g3, stbl, acc,
                gs0, gs1, gs2, gs3, ss0, ss1, ss2, ss3):
    gb = [g0, g1, g2, g3]
    gs = [gs0, gs1, gs2, gs3]
    ss = [ss0, ss1, ss2, ss3]
    c = lax.axis_index("c")
    s = lax.axis_index("s")

    pltpu.sync_copy(tbls.at[c, pl.ds(s * RPT, RPT)],
                    stbl.at[pl.ds(s * RPT, RPT)])

    def zrow(i, _):
        def zlane(j, _):
            g0[i, pl.ds(j * LANES, LANES)] = jnp.zeros((LANES,), jnp.float32)
            return 0

        return lax.fori_loop(0, DH // LANES, zlane, 0)

    lax.fori_loop(0, K, zrow, 0)

    def zcp(t, _):
        pltpu.sync_copy(g0, acc.at[pl.ds(s * RPT + t * K, K)])
        return 0

    lax.fori_loop(0, RPT // K, zcp, 0)
    plsc.subcore_barrier()

    def gd(j, t):
        return pltpu.make_async_copy(stbl.at[src_v.at[j]], gb[t], gs[t])

    def sd(j, t):
        return pltpu.make_async_copy(gb[t], acc.at[dst_v.at[j]], ss[t])

    for h in range(4):
        base = s * 2 * NCHUNK + h * HCH
        pltpu.sync_copy(src_r.at[pl.ds(base, HCH)], src_v)
        pltpu.sync_copy(dst_r.at[pl.ds(base, HCH)], dst_v)
        gd(0, 0).start()
        gd(1, 1).start()

        def step(i, _):
            for t in range(4):
                j = i * 4 + t
                gd(j, t).wait()
                sd(j, t).start(add=True)
                t2 = (t + 2) % 4

                @pl.when(j + 2 < HCH)
                def _next():
                    @pl.when(j >= 2)
                    def _drain():
                        sd(j - 2, t2).wait()

                    gd(j + 2, t2).start()

            return 0

        lax.fori_loop(0, HCH // 4, step, 0)
        for j in range(HCH - 4, HCH):
            sd(j, j % 4).wait()

    plsc.subcore_barrier()
    pltpu.sync_copy(acc.at[pl.ds(s * RPT, RPT)],
                    out.at[c, pl.ds(s * RPT, RPT)])


_msgab_kernel = functools.partial(
    pl.kernel,
    out_type=jax.ShapeDtypeStruct((NC, N_PAD, DH), jnp.float32),
    mesh=_MESH,
    scratch_types=_msg_scratch(DH),
    compiler_params=pltpu.CompilerParams(use_tc_tiling_on_sc=False),
)(_msgab_body)


BT = 2048


def _norm_scale_body(x_ref, dego_ref, degi_ref, xsab_ref, ns_ref, nd_ref):
    no_r = lax.rsqrt(jnp.maximum(dego_ref[0:1, :] + dego_ref[1:2, :], 1.0))
    nd_r = lax.rsqrt(jnp.maximum(degi_ref[0:1, :] + degi_ref[1:2, :], 1.0))
    no = jnp.transpose(no_r)
    nd = jnp.transpose(nd_r)
    xs = x_ref[...] * no
    xsab_ref[0] = xs[:, :DH]
    xsab_ref[1] = xs[:, DH:]
    ns_ref[...] = no
    nd_ref[...] = nd


def _norm_scale(x_pad, dego, degi):
    grid = (N_PAD // BT,)
    return pl.pallas_call(
        _norm_scale_body,
        grid=grid,
        in_specs=[
            pl.BlockSpec((BT, D_IN), lambda i: (i, 0)),
            pl.BlockSpec((NC, BT), lambda i: (0, i)),
            pl.BlockSpec((NC, BT), lambda i: (0, i)),
        ],
        out_specs=[
            pl.BlockSpec((NC, BT, DH), lambda i: (0, i, 0)),
            pl.BlockSpec((BT, 1), lambda i: (i, 0)),
            pl.BlockSpec((BT, 1), lambda i: (i, 0)),
        ],
        out_shape=[
            jax.ShapeDtypeStruct((NC, N_PAD, DH), jnp.float32),
            jax.ShapeDtypeStruct((N_PAD, 1), jnp.float32),
            jax.ShapeDtypeStruct((N_PAD, 1), jnp.float32),
        ],
    )(x_pad, dego, degi)


def _layer_body(agg_ref, w1_ref, b1_ref, ns_ref, nd_ref, w2_ref, m2_ref):
    h = (jnp.dot(agg_ref[0], w1_ref[:DH, :],
                 preferred_element_type=jnp.float32)
         + jnp.dot(agg_ref[1], w1_ref[DH:, :],
                   preferred_element_type=jnp.float32))
    h = jnp.maximum(h * nd_ref[...] + b1_ref[...], 0.0)
    m2_ref[...] = jnp.dot(h * ns_ref[...], w2_ref[...],
                          preferred_element_type=jnp.float32)


def _layer(agg, w1, b1, ns, nd, w2p):
    grid = (N_PAD // BT,)
    return pl.pallas_call(
        _layer_body,
        grid=grid,
        in_specs=[
            pl.BlockSpec((NC, BT, DH), lambda i: (0, i, 0)),
            pl.BlockSpec((D_IN, D_H), lambda i: (0, 0)),
            pl.BlockSpec((1, D_H), lambda i: (0, 0)),
            pl.BlockSpec((BT, 1), lambda i: (i, 0)),
            pl.BlockSpec((BT, 1), lambda i: (i, 0)),
            pl.BlockSpec((D_H, D2), lambda i: (0, 0)),
        ],
        out_specs=pl.BlockSpec((BT, D2), lambda i: (i, 0)),
        out_shape=jax.ShapeDtypeStruct((N_PAD, D2), jnp.float32),
    )(agg, w1, b1.reshape(1, D_H), ns, nd, w2p)


def _final_body(agg_ref, nd_ref, b2_ref, out_ref):
    res = (agg_ref[0] + agg_ref[1]) * nd_ref[...] + b2_ref[...]
    out_ref[...] = res[:, :N_CLS]


def _final(agg2, nd, b2p):
    grid = (N_PAD // BT,)
    return pl.pallas_call(
        _final_body,
        grid=grid,
        in_specs=[
            pl.BlockSpec((NC, BT, D2), lambda i: (0, i, 0)),
            pl.BlockSpec((BT, 1), lambda i: (i, 0)),
            pl.BlockSpec((1, D2), lambda i: (0, 0)),
        ],
        out_specs=pl.BlockSpec((BT, N_CLS), lambda i: (i, 0)),
        out_shape=jax.ShapeDtypeStruct((N, N_CLS), jnp.float32),
    )(agg2, nd, b2p)


def kernel(in_feat, edge_index, W1, b1, W2, b2):
    src = edge_index[0]
    dst = edge_index[1]
    pad = jnp.full((E_PAD - E,), PAD_NODE, jnp.int32)
    src_r = jnp.concatenate([src, pad]).reshape(NW * NCHUNK, K)
    dst_r = jnp.concatenate([dst, pad]).reshape(NW * NCHUNK, K)

    x_pad = jnp.pad(in_feat, ((0, N_PAD - N), (0, 0)))
    w2p = jnp.pad(W2, ((0, 0), (0, D2 - N_CLS)))
    b2p = jnp.pad(b2, (0, D2 - N_CLS)).reshape(1, D2)

    dego, degi = _deg_kernel(src_r, dst_r)
    xsab, ns, nd = _norm_scale(x_pad, dego, degi)
    agg1 = _msgab_kernel(xsab, src_r, dst_r)
    m2 = _layer(agg1, W1, b1, ns, nd, w2p)
    agg2 = _msg_kernel(m2, src_r, dst_r)
    return _final(agg2, nd, b2p)


# speedup vs baseline: 12.4392x; 1.0006x over previous
import functools

import jax
import jax.numpy as jnp
from jax import lax
from jax.experimental import pallas as pl
from jax.experimental.pallas import tpu as pltpu
from jax.experimental.pallas import tpu_sc as plsc

N = 10000
E = 320000
D_IN = 128
D_H = 128
N_CLS = 40
DH = 64
D2 = 48

NC = 2
NS = 16
LANES = 16
NW = NC * NS

K = 128
NCHUNK = 80
EPW = NCHUNK * K
E_PAD = EPW * NW

N_PAD = 10240
RPT = N_PAD // NS
PAD_NODE = N_PAD - 1

BN = 256
_MESH = plsc.VectorSubcoreMesh(core_axis_name="c", subcore_axis_name="s")


def _deg_body(src_r, dst_r, dego, degi, src_v, dst_v, ones_v, zbuf,
              sdego, sdegi, semo, semi):
    c = lax.axis_index("c")
    s = lax.axis_index("s")
    wid = s * NC + c

    def fill(i, _):
        ones_v[pl.ds(i * LANES, LANES)] = jnp.ones((LANES,), jnp.float32)
        zbuf[pl.ds(i * LANES, LANES)] = jnp.zeros((LANES,), jnp.float32)
        return 0

    lax.fori_loop(0, K // LANES, fill, 0)

    def zfill(i, _):
        zbuf[pl.ds(K + i * LANES, LANES)] = jnp.zeros((LANES,), jnp.float32)
        return 0

    lax.fori_loop(0, (RPT - K) // LANES, zfill, 0)
    pltpu.sync_copy(zbuf, sdego.at[pl.ds(s * RPT, RPT)])
    pltpu.sync_copy(zbuf, sdegi.at[pl.ds(s * RPT, RPT)])

    pltpu.sync_copy(src_r.at[pl.ds(wid * NCHUNK, NCHUNK)], src_v)
    pltpu.sync_copy(dst_r.at[pl.ds(wid * NCHUNK, NCHUNK)], dst_v)
    plsc.subcore_barrier()

    def sdo(j):
        return pltpu.make_async_copy(ones_v, sdego.at[src_v.at[j]], semo)

    def sdi(j):
        return pltpu.make_async_copy(ones_v, sdegi.at[dst_v.at[j]], semi)

    def step(j, _):
        sdo(j).start(add=True)
        sdi(j).start(add=True)
        return 0

    lax.fori_loop(0, NCHUNK, step, 0)

    def dstep(j, _):
        sdo(j).wait()
        sdi(j).wait()
        return 0

    lax.fori_loop(0, NCHUNK, dstep, 0)
    plsc.subcore_barrier()
    pltpu.sync_copy(sdego.at[pl.ds(s * RPT, RPT)],
                    dego.at[c, pl.ds(s * RPT, RPT)])
    pltpu.sync_copy(sdegi.at[pl.ds(s * RPT, RPT)],
                    degi.at[c, pl.ds(s * RPT, RPT)])


_deg_kernel = functools.partial(
    pl.kernel,
    out_type=(
        jax.ShapeDtypeStruct((NC, N_PAD), jnp.float32),
        jax.ShapeDtypeStruct((NC, N_PAD), jnp.float32),
    ),
    mesh=_MESH,
    scratch_types=[
        pltpu.VMEM((NCHUNK, K), jnp.int32),
        pltpu.VMEM((NCHUNK, K), jnp.int32),
        pltpu.VMEM((K,), jnp.float32),
        pltpu.VMEM((RPT,), jnp.float32),
        pltpu.VMEM_SHARED((N_PAD,), jnp.float32),
        pltpu.VMEM_SHARED((N_PAD,), jnp.float32),
        pltpu.SemaphoreType.DMA,
        pltpu.SemaphoreType.DMA,
    ],
    compiler_params=pltpu.CompilerParams(use_tc_tiling_on_sc=False),
)(_deg_body)


HCH = NCHUNK // 2


def _msg_body(table, src_r, dst_r, out, src_v, dst_v,
              g0, g1, g2, g3, stbl, acc,
              gs0, gs1, gs2, gs3, ss0, ss1, ss2, ss3):
    gb = [g0, g1, g2, g3]
    gs = [gs0, gs1, gs2, gs3]
    ss = [ss0, ss1, ss2, ss3]
    c = lax.axis_index("c")
    s = lax.axis_index("s")
    wid = s * NC + c

    stage = pltpu.make_async_copy(table.at[pl.ds(s * RPT, RPT)],
                                  stbl.at[pl.ds(s * RPT, RPT)], gs0)
    stage.start()

    def zrow(i, _):
        def zlane(j, _):
            g0[i, pl.ds(j * LANES, LANES)] = jnp.zeros((LANES,), jnp.float32)
            return 0

        return lax.fori_loop(0, D2 // LANES, zlane, 0)

    lax.fori_loop(0, K, zrow, 0)

    def zcp(t, _):
        pltpu.sync_copy(g0, acc.at[pl.ds(s * RPT + t * K, K)])
        return 0

    lax.fori_loop(0, RPT // K, zcp, 0)
    stage.wait()
    plsc.subcore_barrier()

    def gd(j, t):
        return pltpu.make_async_copy(stbl.at[src_v.at[j]], gb[t], gs[t])

    def sd(j, t):
        return pltpu.make_async_copy(gb[t], acc.at[dst_v.at[j]], ss[t])

    for h in range(2):
        pltpu.sync_copy(src_r.at[pl.ds(wid * NCHUNK + h * HCH, HCH)], src_v)
        pltpu.sync_copy(dst_r.at[pl.ds(wid * NCHUNK + h * HCH, HCH)], dst_v)
        gd(0, 0).start()
        gd(1, 1).start()

        def step(i, _):
            for t in range(4):
                j = i * 4 + t
                gd(j, t).wait()
                sd(j, t).start(add=True)
                t2 = (t + 2) % 4

                @pl.when(j + 2 < HCH)
                def _next():
                    @pl.when(j >= 2)
                    def _drain():
                        sd(j - 2, t2).wait()

                    gd(j + 2, t2).start()

            return 0

        lax.fori_loop(0, HCH // 4, step, 0)
        for j in range(HCH - 4, HCH):
            sd(j, j % 4).wait()

    plsc.subcore_barrier()
    pltpu.sync_copy(acc.at[pl.ds(s * RPT, RPT)],
                    out.at[c, pl.ds(s * RPT, RPT)])


def _msg_scratch(w):
    return [
        pltpu.VMEM((HCH, K), jnp.int32),
        pltpu.VMEM((HCH, K), jnp.int32),
        pltpu.VMEM((K, w), jnp.float32),
        pltpu.VMEM((K, w), jnp.float32),
        pltpu.VMEM((K, w), jnp.float32),
        pltpu.VMEM((K, w), jnp.float32),
        pltpu.VMEM_SHARED((N_PAD, w), jnp.float32),
        pltpu.VMEM_SHARED((N_PAD, w), jnp.float32),
        pltpu.SemaphoreType.DMA,
        pltpu.SemaphoreType.DMA,
        pltpu.SemaphoreType.DMA,
        pltpu.SemaphoreType.DMA,
        pltpu.SemaphoreType.DMA,
        pltpu.SemaphoreType.DMA,
        pltpu.SemaphoreType.DMA,
        pltpu.SemaphoreType.DMA,
    ]


_msg_kernel = functools.partial(
    pl.kernel,
    out_type=jax.ShapeDtypeStruct((NC, N_PAD, D2), jnp.float32),
    mesh=_MESH,
    scratch_types=_msg_scratch(D2),
    compiler_params=pltpu.CompilerParams(use_tc_tiling_on_sc=False),
)(_msg_body)


def _msgab_body(tbls, src_r, dst_r, out, src_v, dst_v,
                g0, g1, g2, g3, stbl, acc,
                gs0, gs1, gs2, gs3, ss0, ss1, ss2, ss3):
    gb = [g0, g1, g2, g3]
    gs = [gs0, gs1, gs2, gs3]
    ss = [ss0, ss1, ss2, ss3]
    c = lax.axis_index("c")
    s = lax.axis_index("s")

    stage = pltpu.make_async_copy(tbls.at[c, pl.ds(s * RPT, RPT)],
                                  stbl.at[pl.ds(s * RPT, RPT)], gs0)
    stage.start()

    def zrow(i, _):
        def zlane(j, _):
            g0[i, pl.ds(j * LANES, LANES)] = jnp.zeros((LANES,), jnp.float32)
            return 0

        return lax.fori_loop(0, DH // LANES, zlane, 0)

    lax.fori_loop(0, K, zrow, 0)

    def zcp(t, _):
        pltpu.sync_copy(g0, acc.at[pl.ds(s * RPT + t * K, K)])
        return 0

    lax.fori_loop(0, RPT // K, zcp, 0)
    stage.wait()
    plsc.subcore_barrier()

    def gd(j, t):
        return pltpu.make_async_copy(stbl.at[src_v.at[j]], gb[t], gs[t])

    def sd(j, t):
        return pltpu.make_async_copy(gb[t], acc.at[dst_v.at[j]], ss[t])

    for h in range(4):
        base = s * 2 * NCHUNK + h * HCH
        pltpu.sync_copy(src_r.at[pl.ds(base, HCH)], src_v)
        pltpu.sync_copy(dst_r.at[pl.ds(base, HCH)], dst_v)
        gd(0, 0).start()
        gd(1, 1).start()

        def step(i, _):
            for t in range(4):
                j = i * 4 + t
                gd(j, t).wait()
                sd(j, t).start(add=True)
                t2 = (t + 2) % 4

                @pl.when(j + 2 < HCH)
                def _next():
                    @pl.when(j >= 2)
                    def _drain():
                        sd(j - 2, t2).wait()

                    gd(j + 2, t2).start()

            return 0

        lax.fori_loop(0, HCH // 4, step, 0)
        for j in range(HCH - 4, HCH):
            sd(j, j % 4).wait()

    plsc.subcore_barrier()
    pltpu.sync_copy(acc.at[pl.ds(s * RPT, RPT)],
                    out.at[c, pl.ds(s * RPT, RPT)])


_msgab_kernel = functools.partial(
    pl.kernel,
    out_type=jax.ShapeDtypeStruct((NC, N_PAD, DH), jnp.float32),
    mesh=_MESH,
    scratch_types=_msg_scratch(DH),
    compiler_params=pltpu.CompilerParams(use_tc_tiling_on_sc=False),
)(_msgab_body)


BT = 2048


def _norm_scale_body(x_ref, dego_ref, degi_ref, xsab_ref, ns_ref, nd_ref):
    no_r = lax.rsqrt(jnp.maximum(dego_ref[0:1, :] + dego_ref[1:2, :], 1.0))
    nd_r = lax.rsqrt(jnp.maximum(degi_ref[0:1, :] + degi_ref[1:2, :], 1.0))
    no = jnp.transpose(no_r)
    nd = jnp.transpose(nd_r)
    xs = x_ref[...] * no
    xsab_ref[0] = xs[:, :DH]
    xsab_ref[1] = xs[:, DH:]
    ns_ref[...] = no
    nd_ref[...] = nd


def _norm_scale(x_pad, dego, degi):
    grid = (N_PAD // BT,)
    return pl.pallas_call(
        _norm_scale_body,
        grid=grid,
        in_specs=[
            pl.BlockSpec((BT, D_IN), lambda i: (i, 0)),
            pl.BlockSpec((NC, BT), lambda i: (0, i)),
            pl.BlockSpec((NC, BT), lambda i: (0, i)),
        ],
        out_specs=[
            pl.BlockSpec((NC, BT, DH), lambda i: (0, i, 0)),
            pl.BlockSpec((BT, 1), lambda i: (i, 0)),
            pl.BlockSpec((BT, 1), lambda i: (i, 0)),
        ],
        out_shape=[
            jax.ShapeDtypeStruct((NC, N_PAD, DH), jnp.float32),
            jax.ShapeDtypeStruct((N_PAD, 1), jnp.float32),
            jax.ShapeDtypeStruct((N_PAD, 1), jnp.float32),
        ],
    )(x_pad, dego, degi)


def _layer_body(agg_ref, w1_ref, b1_ref, ns_ref, nd_ref, w2_ref, m2_ref):
    h = (jnp.dot(agg_ref[0], w1_ref[:DH, :],
                 preferred_element_type=jnp.float32)
         + jnp.dot(agg_ref[1], w1_ref[DH:, :],
                   preferred_element_type=jnp.float32))
    h = jnp.maximum(h * nd_ref[...] + b1_ref[...], 0.0)
    m2_ref[...] = jnp.dot(h * ns_ref[...], w2_ref[...],
                          preferred_element_type=jnp.float32)


def _layer(agg, w1, b1, ns, nd, w2p):
    grid = (N_PAD // BT,)
    return pl.pallas_call(
        _layer_body,
        grid=grid,
        in_specs=[
            pl.BlockSpec((NC, BT, DH), lambda i: (0, i, 0)),
            pl.BlockSpec((D_IN, D_H), lambda i: (0, 0)),
            pl.BlockSpec((1, D_H), lambda i: (0, 0)),
            pl.BlockSpec((BT, 1), lambda i: (i, 0)),
            pl.BlockSpec((BT, 1), lambda i: (i, 0)),
            pl.BlockSpec((D_H, D2), lambda i: (0, 0)),
        ],
        out_specs=pl.BlockSpec((BT, D2), lambda i: (i, 0)),
        out_shape=jax.ShapeDtypeStruct((N_PAD, D2), jnp.float32),
    )(agg, w1, b1.reshape(1, D_H), ns, nd, w2p)


def _final_body(agg_ref, nd_ref, b2_ref, out_ref):
    res = (agg_ref[0] + agg_ref[1]) * nd_ref[...] + b2_ref[...]
    out_ref[...] = res[:, :N_CLS]


def _final(agg2, nd, b2p):
    grid = (N_PAD // BT,)
    return pl.pallas_call(
        _final_body,
        grid=grid,
        in_specs=[
            pl.BlockSpec((NC, BT, D2), lambda i: (0, i, 0)),
            pl.BlockSpec((BT, 1), lambda i: (i, 0)),
            pl.BlockSpec((1, D2), lambda i: (0, 0)),
        ],
        out_specs=pl.BlockSpec((BT, N_CLS), lambda i: (i, 0)),
        out_shape=jax.ShapeDtypeStruct((N, N_CLS), jnp.float32),
    )(agg2, nd, b2p)


def kernel(in_feat, edge_index, W1, b1, W2, b2):
    src = edge_index[0]
    dst = edge_index[1]
    pad = jnp.full((E_PAD - E,), PAD_NODE, jnp.int32)
    src_r = jnp.concatenate([src, pad]).reshape(NW * NCHUNK, K)
    dst_r = jnp.concatenate([dst, pad]).reshape(NW * NCHUNK, K)

    x_pad = jnp.pad(in_feat, ((0, N_PAD - N), (0, 0)))
    w2p = jnp.pad(W2, ((0, 0), (0, D2 - N_CLS)))
    b2p = jnp.pad(b2, (0, D2 - N_CLS)).reshape(1, D2)

    dego, degi = _deg_kernel(src_r, dst_r)
    xsab, ns, nd = _norm_scale(x_pad, dego, degi)
    agg1 = _msgab_kernel(xsab, src_r, dst_r)
    m2 = _layer(agg1, W1, b1, ns, nd, w2p)
    agg2 = _msg_kernel(m2, src_r, dst_r)
    return _final(agg2, nd, b2p)


# speedup vs baseline: 12.5566x; 1.0094x over previous
import functools

import jax
import jax.numpy as jnp
from jax import lax
from jax.experimental import pallas as pl
from jax.experimental.pallas import tpu as pltpu
from jax.experimental.pallas import tpu_sc as plsc

N = 10000
E = 320000
D_IN = 128
D_H = 128
N_CLS = 40
DH = 64
D2 = 48

NC = 2
NS = 16
LANES = 16
NW = NC * NS

K = 128
NCHUNK = 80
EPW = NCHUNK * K
E_PAD = EPW * NW

N_PAD = 10240
RPT = N_PAD // NS
PAD_NODE = N_PAD - 1

_MESH = plsc.VectorSubcoreMesh(core_axis_name="c", subcore_axis_name="s")


def _deg_body(src_r, dst_r, dego, degi, src_v, dst_v, ones_v, zbuf,
              sdego, sdegi, semo, semi):
    c = lax.axis_index("c")
    s = lax.axis_index("s")
    wid = s * NC + c

    def fill(i, _):
        ones_v[pl.ds(i * LANES, LANES)] = jnp.ones((LANES,), jnp.float32)
        zbuf[pl.ds(i * LANES, LANES)] = jnp.zeros((LANES,), jnp.float32)
        return 0

    lax.fori_loop(0, K // LANES, fill, 0)

    def zfill(i, _):
        zbuf[pl.ds(K + i * LANES, LANES)] = jnp.zeros((LANES,), jnp.float32)
        return 0

    lax.fori_loop(0, (RPT - K) // LANES, zfill, 0)
    pltpu.sync_copy(zbuf, sdego.at[pl.ds(s * RPT, RPT)])
    pltpu.sync_copy(zbuf, sdegi.at[pl.ds(s * RPT, RPT)])

    pltpu.sync_copy(src_r.at[pl.ds(wid * NCHUNK, NCHUNK)], src_v)
    pltpu.sync_copy(dst_r.at[pl.ds(wid * NCHUNK, NCHUNK)], dst_v)
    plsc.subcore_barrier()

    def sdo(j):
        return pltpu.make_async_copy(ones_v, sdego.at[src_v.at[j]], semo)

    def sdi(j):
        return pltpu.make_async_copy(ones_v, sdegi.at[dst_v.at[j]], semi)

    def step(j, _):
        sdo(j).start(add=True)
        sdi(j).start(add=True)
        return 0

    lax.fori_loop(0, NCHUNK, step, 0)

    def dstep(j, _):
        sdo(j).wait()
        sdi(j).wait()
        return 0

    lax.fori_loop(0, NCHUNK, dstep, 0)
    plsc.subcore_barrier()
    pltpu.sync_copy(sdego.at[pl.ds(s * RPT, RPT)],
                    dego.at[c, pl.ds(s * RPT, RPT)])
    pltpu.sync_copy(sdegi.at[pl.ds(s * RPT, RPT)],
                    degi.at[c, pl.ds(s * RPT, RPT)])


_deg_kernel = functools.partial(
    pl.kernel,
    out_type=(
        jax.ShapeDtypeStruct((NC, N_PAD), jnp.float32),
        jax.ShapeDtypeStruct((NC, N_PAD), jnp.float32),
    ),
    mesh=_MESH,
    scratch_types=[
        pltpu.VMEM((NCHUNK, K), jnp.int32),
        pltpu.VMEM((NCHUNK, K), jnp.int32),
        pltpu.VMEM((K,), jnp.float32),
        pltpu.VMEM((RPT,), jnp.float32),
        pltpu.VMEM_SHARED((N_PAD,), jnp.float32),
        pltpu.VMEM_SHARED((N_PAD,), jnp.float32),
        pltpu.SemaphoreType.DMA,
        pltpu.SemaphoreType.DMA,
    ],
    compiler_params=pltpu.CompilerParams(use_tc_tiling_on_sc=False),
)(_deg_body)


HCH = NCHUNK // 2


def _msg_body(table, src_r, dst_r, out, src_v, dst_v,
              g0, g1, g2, g3, stbl, acc,
              gs0, gs1, gs2, gs3, ss0, ss1, ss2, ss3):
    gb = [g0, g1, g2, g3]
    gs = [gs0, gs1, gs2, gs3]
    ss = [ss0, ss1, ss2, ss3]
    c = lax.axis_index("c")
    s = lax.axis_index("s")
    wid = s * NC + c

    stage = pltpu.make_async_copy(table.at[pl.ds(s * RPT, RPT)],
                                  stbl.at[pl.ds(s * RPT, RPT)], gs0)
    stage.start()

    def zrow(i, _):
        def zlane(j, _):
            g0[i, pl.ds(j * LANES, LANES)] = jnp.zeros((LANES,), jnp.float32)
            return 0

        return lax.fori_loop(0, D2 // LANES, zlane, 0)

    lax.fori_loop(0, K, zrow, 0)

    def zcp(t, _):
        pltpu.sync_copy(g0, acc.at[pl.ds(s * RPT + t * K, K)])
        return 0

    lax.fori_loop(0, RPT // K, zcp, 0)
    stage.wait()
    plsc.subcore_barrier()

    def gd(j, t):
        return pltpu.make_async_copy(stbl.at[src_v.at[j]], gb[t], gs[t])

    def sd(j, t):
        return pltpu.make_async_copy(gb[t], acc.at[dst_v.at[j]], ss[t])

    for h in range(2):
        pltpu.sync_copy(src_r.at[pl.ds(wid * NCHUNK + h * HCH, HCH)], src_v)
        pltpu.sync_copy(dst_r.at[pl.ds(wid * NCHUNK + h * HCH, HCH)], dst_v)
        gd(0, 0).start()
        gd(1, 1).start()

        def step(i, _):
            for t in range(4):
                j = i * 4 + t
                gd(j, t).wait()
                sd(j, t).start(add=True)
                t2 = (t + 2) % 4

                @pl.when(j + 2 < HCH)
                def _next():
                    @pl.when(j >= 2)
                    def _drain():
                        sd(j - 2, t2).wait()

                    gd(j + 2, t2).start()

            return 0

        lax.fori_loop(0, HCH // 4, step, 0)
        for j in range(HCH - 4, HCH):
            sd(j, j % 4).wait()

    plsc.subcore_barrier()
    pltpu.sync_copy(acc.at[pl.ds(s * RPT, RPT)],
                    out.at[c, pl.ds(s * RPT, RPT)])


def _msg_scratch(w):
    return [
        pltpu.VMEM((HCH, K), jnp.int32),
        pltpu.VMEM((HCH, K), jnp.int32),
        pltpu.VMEM((K, w), jnp.float32),
        pltpu.VMEM((K, w), jnp.float32),
        pltpu.VMEM((K, w), jnp.float32),
        pltpu.VMEM((K, w), jnp.float32),
        pltpu.VMEM_SHARED((N_PAD, w), jnp.float32),
        pltpu.VMEM_SHARED((N_PAD, w), jnp.float32),
        pltpu.SemaphoreType.DMA,
        pltpu.SemaphoreType.DMA,
        pltpu.SemaphoreType.DMA,
        pltpu.SemaphoreType.DMA,
        pltpu.SemaphoreType.DMA,
        pltpu.SemaphoreType.DMA,
        pltpu.SemaphoreType.DMA,
        pltpu.SemaphoreType.DMA,
    ]


_msg_kernel = functools.partial(
    pl.kernel,
    out_type=jax.ShapeDtypeStruct((NC, N_PAD, D2), jnp.float32),
    mesh=_MESH,
    scratch_types=_msg_scratch(D2),
    compiler_params=pltpu.CompilerParams(use_tc_tiling_on_sc=False),
)(_msg_body)


def _msgab_body(tbls, src_r, dst_r, out, src_v, dst_v,
                g0, g1, g2, g3, stbl, acc,
                gs0, gs1, gs2, gs3, ss0, ss1, ss2, ss3):
    gb = [g0, g1, g2, g3]
    gs = [gs0, gs1, gs2, gs3]
    ss = [ss0, ss1, ss2, ss3]
    c = lax.axis_index("c")
    s = lax.axis_index("s")

    stage = pltpu.make_async_copy(tbls.at[c, pl.ds(s * RPT, RPT)],
                                  stbl.at[pl.ds(s * RPT, RPT)], gs0)
    stage.start()

    def zrow(i, _):
        def zlane(j, _):
            g0[i, pl.ds(j * LANES, LANES)] = jnp.zeros((LANES,), jnp.float32)
            return 0

        return lax.fori_loop(0, DH // LANES, zlane, 0)

    lax.fori_loop(0, K, zrow, 0)

    def zcp(t, _):
        pltpu.sync_copy(g0, acc.at[pl.ds(s * RPT + t * K, K)])
        return 0

    lax.fori_loop(0, RPT // K, zcp, 0)
    stage.wait()
    plsc.subcore_barrier()

    def gd(j, t):
        return pltpu.make_async_copy(stbl.at[src_v.at[j]], gb[t], gs[t])

    def sd(j, t):
        return pltpu.make_async_copy(gb[t], acc.at[dst_v.at[j]], ss[t])

    for h in range(4):
        base = s * 2 * NCHUNK + h * HCH
        pltpu.sync_copy(src_r.at[pl.ds(base, HCH)], src_v)
        pltpu.sync_copy(dst_r.at[pl.ds(base, HCH)], dst_v)
        gd(0, 0).start()
        gd(1, 1).start()

        def step(i, _):
            for t in range(4):
                j = i * 4 + t
                gd(j, t).wait()
                sd(j, t).start(add=True)
                t2 = (t + 2) % 4

                @pl.when(j + 2 < HCH)
                def _next():
                    @pl.when(j >= 2)
                    def _drain():
                        sd(j - 2, t2).wait()

                    gd(j + 2, t2).start()

            return 0

        lax.fori_loop(0, HCH // 4, step, 0)
        for j in range(HCH - 4, HCH):
            sd(j, j % 4).wait()

    plsc.subcore_barrier()
    pltpu.sync_copy(acc.at[pl.ds(s * RPT, RPT)],
                    out.at[c, pl.ds(s * RPT, RPT)])


_msgab_kernel = functools.partial(
    pl.kernel,
    out_type=jax.ShapeDtypeStruct((NC, N_PAD, DH), jnp.float32),
    mesh=_MESH,
    scratch_types=_msg_scratch(DH),
    compiler_params=pltpu.CompilerParams(use_tc_tiling_on_sc=False),
)(_msgab_body)


BT = 2048


def _norm_scale_body(x_ref, dego_ref, degi_ref, xsab_ref, ns_ref, nd_ref):
    no_r = lax.rsqrt(jnp.maximum(dego_ref[0:1, :] + dego_ref[1:2, :], 1.0))
    nd_r = lax.rsqrt(jnp.maximum(degi_ref[0:1, :] + degi_ref[1:2, :], 1.0))
    no = jnp.transpose(no_r)
    nd = jnp.transpose(nd_r)
    xs = x_ref[...] * no
    xsab_ref[0] = xs[:, :DH]
    xsab_ref[1] = xs[:, DH:]
    ns_ref[...] = no
    nd_ref[...] = nd


def _norm_scale(x_pad, dego, degi):
    grid = (N_PAD // BT,)
    return pl.pallas_call(
        _norm_scale_body,
        grid=grid,
        in_specs=[
            pl.BlockSpec((BT, D_IN), lambda i: (i, 0)),
            pl.BlockSpec((NC, BT), lambda i: (0, i)),
            pl.BlockSpec((NC, BT), lambda i: (0, i)),
        ],
        out_specs=[
            pl.BlockSpec((NC, BT, DH), lambda i: (0, i, 0)),
            pl.BlockSpec((BT, 1), lambda i: (i, 0)),
            pl.BlockSpec((BT, 1), lambda i: (i, 0)),
        ],
        out_shape=[
            jax.ShapeDtypeStruct((NC, N_PAD, DH), jnp.float32),
            jax.ShapeDtypeStruct((N_PAD, 1), jnp.float32),
            jax.ShapeDtypeStruct((N_PAD, 1), jnp.float32),
        ],
    )(x_pad, dego, degi)


def _layer_body(agg_ref, w1_ref, b1_ref, ns_ref, nd_ref, w2_ref, m2_ref):
    h = (jnp.dot(agg_ref[0], w1_ref[:DH, :],
                 preferred_element_type=jnp.float32)
         + jnp.dot(agg_ref[1], w1_ref[DH:, :],
                   preferred_element_type=jnp.float32))
    h = jnp.maximum(h * nd_ref[...] + b1_ref[...], 0.0)
    m2_ref[...] = jnp.dot(h * ns_ref[...], w2_ref[...],
                          preferred_element_type=jnp.float32)


def _layer(agg, w1, b1, ns, nd, w2p):
    grid = (N_PAD // BT,)
    return pl.pallas_call(
        _layer_body,
        grid=grid,
        in_specs=[
            pl.BlockSpec((NC, BT, DH), lambda i: (0, i, 0)),
            pl.BlockSpec((D_IN, D_H), lambda i: (0, 0)),
            pl.BlockSpec((1, D_H), lambda i: (0, 0)),
            pl.BlockSpec((BT, 1), lambda i: (i, 0)),
            pl.BlockSpec((BT, 1), lambda i: (i, 0)),
            pl.BlockSpec((D_H, D2), lambda i: (0, 0)),
        ],
        out_specs=pl.BlockSpec((BT, D2), lambda i: (i, 0)),
        out_shape=jax.ShapeDtypeStruct((N_PAD, D2), jnp.float32),
    )(agg, w1, b1.reshape(1, D_H), ns, nd, w2p)


def _final_body(agg_ref, nd_ref, b2_ref, out_ref):
    res = (agg_ref[0] + agg_ref[1]) * nd_ref[...] + b2_ref[...]
    out_ref[...] = res[:, :N_CLS]


def _final(agg2, nd, b2p):
    grid = (N_PAD // BT,)
    return pl.pallas_call(
        _final_body,
        grid=grid,
        in_specs=[
            pl.BlockSpec((NC, BT, D2), lambda i: (0, i, 0)),
            pl.BlockSpec((BT, 1), lambda i: (i, 0)),
            pl.BlockSpec((1, D2), lambda i: (0, 0)),
        ],
        out_specs=pl.BlockSpec((BT, N_CLS), lambda i: (i, 0)),
        out_shape=jax.ShapeDtypeStruct((N, N_CLS), jnp.float32),
    )(agg2, nd, b2p)


def kernel(in_feat, edge_index, W1, b1, W2, b2):
    src = edge_index[0]
    dst = edge_index[1]
    pad = jnp.full((E_PAD - E,), PAD_NODE, jnp.int32)
    src_r = jnp.concatenate([src, pad]).reshape(NW * NCHUNK, K)
    dst_r = jnp.concatenate([dst, pad]).reshape(NW * NCHUNK, K)

    x_pad = jnp.pad(in_feat, ((0, N_PAD - N), (0, 0)))
    w2p = jnp.pad(W2, ((0, 0), (0, D2 - N_CLS)))
    b2p = jnp.pad(b2, (0, D2 - N_CLS)).reshape(1, D2)

    dego, degi = _deg_kernel(src_r, dst_r)
    xsab, ns, nd = _norm_scale(x_pad, dego, degi)
    agg1 = _msgab_kernel(xsab, src_r, dst_r)
    m2 = _layer(agg1, W1, b1, ns, nd, w2p)
    agg2 = _msg_kernel(m2, src_r, dst_r)
    return _final(agg2, nd, b2p)
